# Initial kernel scaffold; baseline (speedup 1.0000x reference)
import functools

import jax
import jax.numpy as jnp
from jax import lax
from jax.experimental import pallas as pl
from jax.experimental.pallas import tpu as pltpu
from jax.experimental.pallas import tpu_sc as plsc

N_NODES = 10000
H = 32
NC = 2
NS = 16
NW = NC * NS
EB = 128
NPAD = 10240
ROWS_PER_TILE = NPAD // NS


def _zero_f32_vec(ref, n):
    z = jnp.zeros((16,), jnp.float32)

    def body(i, _):
        ref[pl.ds(i * 16, 16)] = z
        return 0

    lax.fori_loop(0, n // 16, body, 0)


def _zero_f32_rows(ref, rows):
    z = jnp.zeros((16,), jnp.float32)

    def body(r, _):
        ref[r, pl.ds(0, 16)] = z
        ref[r, pl.ds(16, 16)] = z
        return 0

    lax.fori_loop(0, rows, body, 0)


def _make_deg_kernel(kb):
    mesh = plsc.VectorSubcoreMesh(core_axis_name="c", subcore_axis_name="s")

    @functools.partial(
        pl.kernel,
        mesh=mesh,
        out_type=jax.ShapeDtypeStruct((NC, NPAD), jnp.float32),
        scratch_types=[
            pltpu.VMEM_SHARED((NPAD,), jnp.float32),
            pltpu.VMEM((kb, EB), jnp.int32),
            pltpu.VMEM((EB,), jnp.float32),
            pltpu.VMEM((EB,), jnp.float32),
        ],
    )
    def deg_kernel(dst_hbm, out_hbm, acc_sh, dst_v, ones_v, tmp_v):
        cid = lax.axis_index("c")
        sid = lax.axis_index("s")
        wid = cid * NS + sid
        one = jnp.ones((16,), jnp.float32)
        for g in range(EB // 16):
            ones_v[pl.ds(g * 16, 16)] = one
        _zero_f32_vec(tmp_v, EB)
        base = sid * ROWS_PER_TILE
        for r in range(ROWS_PER_TILE // EB):
            pltpu.sync_copy(tmp_v, acc_sh.at[pl.ds(base + r * EB, EB)])
        pltpu.sync_copy(dst_hbm.at[wid], dst_v)
        plsc.subcore_barrier()

        def body(j, _):
            pltpu.sync_copy(ones_v, acc_sh.at[dst_v.at[j]], add=True)
            return 0

        lax.fori_loop(0, kb, body, 0)
        plsc.subcore_barrier()
        for r in range(ROWS_PER_TILE // EB):
            off = base + r * EB
            pltpu.sync_copy(acc_sh.at[pl.ds(off, EB)], tmp_v)
            pltpu.sync_copy(tmp_v, out_hbm.at[cid].at[pl.ds(off, EB)])

    return deg_kernel


def _make_agg_kernel(kb, do_c):
    mesh = plsc.VectorSubcoreMesh(core_axis_name="c", subcore_axis_name="s")
    out_type = [jax.ShapeDtypeStruct((NC, NPAD, H), jnp.float32)]
    scratch = [
        pltpu.VMEM_SHARED((NPAD, H), jnp.float32),
        pltpu.VMEM((kb, EB), jnp.int32),
        pltpu.VMEM((kb, EB), jnp.int32),
        pltpu.VMEM((EB, H), jnp.float32),
        pltpu.SemaphoreType.DMA,
    ]
    if do_c:
        out_type.append(jax.ShapeDtypeStruct((NC, NPAD), jnp.float32))
        scratch += [
            pltpu.VMEM_SHARED((NPAD,), jnp.float32),
            pltpu.VMEM((NPAD,), jnp.float32),
            pltpu.VMEM((EB,), jnp.float32),
        ]

    def body_fn(*refs):
        if do_c:
            (y_hbm, src_hbm, dst_hbm, dinv_hbm, p_hbm, c_hbm,
             acc_sh, src_v, dst_v, rows_v, sem, c_sh, dinv_v, val_v) = refs
        else:
            (y_hbm, src_hbm, dst_hbm, p_hbm,
             acc_sh, src_v, dst_v, rows_v, sem) = refs
        cid = lax.axis_index("c")
        sid = lax.axis_index("s")
        wid = cid * NS + sid
        base = sid * ROWS_PER_TILE
        _zero_f32_rows(rows_v, EB)
        for r in range(ROWS_PER_TILE // EB):
            pltpu.sync_copy(rows_v, acc_sh.at[pl.ds(base + r * EB, EB)])
        if do_c:
            _zero_f32_vec(val_v, EB)
            for r in range(ROWS_PER_TILE // EB):
                pltpu.sync_copy(val_v, c_sh.at[pl.ds(base + r * EB, EB)])
            pltpu.sync_copy(dinv_hbm, dinv_v)
        pltpu.sync_copy(src_hbm.at[wid], src_v)
        pltpu.sync_copy(dst_hbm.at[wid], dst_v)
        plsc.subcore_barrier()

        def body(j, _):
            pltpu.async_copy(y_hbm.at[src_v.at[j]], rows_v, sem).wait()
            pltpu.sync_copy(rows_v, acc_sh.at[dst_v.at[j]], add=True)
            if do_c:
                for g in range(EB // 16):
                    didx = dst_v[j, pl.ds(g * 16, 16)]
                    val_v[pl.ds(g * 16, 16)] = plsc.load_gather(dinv_v, [didx])
                pltpu.sync_copy(val_v, c_sh.at[src_v.at[j]], add=True)
            return 0

        lax.fori_loop(0, kb, body, 0)
        plsc.subcore_barrier()
        for r in range(ROWS_PER_TILE // EB):
            off = base + r * EB
            pltpu.sync_copy(acc_sh.at[pl.ds(off, EB)], rows_v)
            pltpu.sync_copy(rows_v, p_hbm.at[cid].at[pl.ds(off, EB)])
        if do_c:
            for r in range(ROWS_PER_TILE // EB):
                off = base + r * EB
                pltpu.sync_copy(c_sh.at[pl.ds(off, EB)], val_v)
                pltpu.sync_copy(val_v, c_hbm.at[cid].at[pl.ds(off, EB)])

    return pl.kernel(body_fn, mesh=mesh, out_type=out_type,
                     scratch_types=scratch)


def _tc_k1(degp_ref, x_ref, w1_ref, dinv_ref, y1_ref):
    iota = lax.broadcasted_iota(jnp.int32, (NPAD, 1), 0)
    valid = iota < N_NODES
    deg = degp_ref[0] + degp_ref[1] + 1.0
    dinv = jnp.where(valid, lax.rsqrt(deg), 0.0)
    dinv_ref[...] = dinv
    xw = jnp.dot(x_ref[...], w1_ref[...], preferred_element_type=jnp.float32)
    y1_ref[...] = dinv * xw


def _tc_k2(p_ref, y_ref, dinv_ref, b_ref, w_ref, out_ref):
    dinv = dinv_ref[...]
    h = jax.nn.relu(dinv * (p_ref[0] + p_ref[1] + y_ref[...]) + b_ref[...])
    out_ref[...] = dinv * jnp.dot(h, w_ref[...],
                                  preferred_element_type=jnp.float32)


def _tc_k3(p_ref, y_ref, dinv_ref, b2_ref, cp_ref, w3_ref, b3_ref,
           lw_ref, lb_ref, out_ref):
    dinv = dinv_ref[...]
    h2 = jax.nn.relu(dinv * (p_ref[0] + p_ref[1] + y_ref[...]) + b2_ref[...])
    c = dinv * (cp_ref[0] + cp_ref[1] + dinv)
    ws = jnp.sum(c * h2, axis=0, keepdims=True)
    pooled = jnp.dot(ws, w3_ref[...],
                     preferred_element_type=jnp.float32) * (1.0 / N_NODES)
    pooled = pooled + b3_ref[...]
    logits = jnp.dot(pooled, lw_ref[...],
                     preferred_element_type=jnp.float32) + lb_ref[...]
    out_ref[...] = jax.nn.softmax(logits, axis=1)


def kernel(x, edge_index, W1, b1, W2, b2, W3, b3, lin_W, lin_b):
    n_edges = edge_index.shape[1]
    epad = -(-n_edges // (NW * EB)) * (NW * EB)
    kb = epad // (NW * EB)

    src = edge_index[0].astype(jnp.int32)
    dst = edge_index[1].astype(jnp.int32)
    pad_cfg = ((0, epad - n_edges),)
    srcr = jnp.pad(src, pad_cfg, constant_values=N_NODES).reshape(NW, kb, EB)
    dstr = jnp.pad(dst, pad_cfg, constant_values=N_NODES).reshape(NW, kb, EB)
    xpad = jnp.pad(x, ((0, NPAD - N_NODES), (0, 0)))

    degp = _make_deg_kernel(kb)(dstr)

    dinv_c, y1 = pl.pallas_call(
        _tc_k1,
        out_shape=(
            jax.ShapeDtypeStruct((NPAD, 1), jnp.float32),
            jax.ShapeDtypeStruct((NPAD, H), jnp.float32),
        ),
    )(degp.reshape(NC, NPAD, 1), xpad, W1)
    dinv_flat = dinv_c.reshape(NPAD)

    p1, cpart = _make_agg_kernel(kb, do_c=True)(y1, srcr, dstr, dinv_flat)

    y2 = pl.pallas_call(
        _tc_k2,
        out_shape=jax.ShapeDtypeStruct((NPAD, H), jnp.float32),
    )(p1, y1, dinv_c, b1.reshape(1, H), W2)

    p2 = _make_agg_kernel(kb, do_c=False)(y2, srcr, dstr)

    out = pl.pallas_call(
        _tc_k3,
        out_shape=jax.ShapeDtypeStruct((1, lin_W.shape[1]), jnp.float32),
    )(p2, y2, dinv_c, b2.reshape(1, H), cpart.reshape(NC, NPAD, 1), W3,
      b3.reshape(1, H), lin_W, lin_b.reshape(1, lin_W.shape[1]))
    return out


# speedup vs baseline: 33.8391x; 33.8391x over previous
import functools

import jax
import jax.numpy as jnp
from jax import lax
from jax.experimental import pallas as pl
from jax.experimental.pallas import tpu as pltpu
from jax.experimental.pallas import tpu_sc as plsc

N_NODES = 10000
H = 32
NC = 2
NS = 16
NW = NC * NS
EB = 128
NPAD = 10240
ROWS_PER_TILE = NPAD // NS


def _zero_f32_vec(ref, n):
    z = jnp.zeros((16,), jnp.float32)

    def body(i, _):
        ref[pl.ds(i * 16, 16)] = z
        return 0

    lax.fori_loop(0, n // 16, body, 0)


def _zero_f32_rows(ref, rows):
    z = jnp.zeros((16,), jnp.float32)

    def body(r, _):
        ref[r, pl.ds(0, 16)] = z
        ref[r, pl.ds(16, 16)] = z
        return 0

    lax.fori_loop(0, rows, body, 0)


def _make_deg_kernel(kb):
    mesh = plsc.VectorSubcoreMesh(core_axis_name="c", subcore_axis_name="s", num_cores=NC, num_subcores=NS)

    @functools.partial(
        pl.kernel,
        mesh=mesh,
        out_type=jax.ShapeDtypeStruct((NC, NPAD), jnp.float32),
        scratch_types=[
            pltpu.VMEM_SHARED((NPAD,), jnp.float32),
            pltpu.VMEM((kb, EB), jnp.int32),
            pltpu.VMEM((EB,), jnp.float32),
            pltpu.VMEM((EB,), jnp.float32),
        ],
    )
    def deg_kernel(dst_hbm, out_hbm, acc_sh, dst_v, ones_v, tmp_v):
        cid = lax.axis_index("c")
        sid = lax.axis_index("s")
        wid = cid * NS + sid
        one = jnp.ones((16,), jnp.float32)
        for g in range(EB // 16):
            ones_v[pl.ds(g * 16, 16)] = one
        _zero_f32_vec(tmp_v, EB)
        base = sid * ROWS_PER_TILE
        for r in range(ROWS_PER_TILE // EB):
            pltpu.sync_copy(tmp_v, acc_sh.at[pl.ds(base + r * EB, EB)])
        pltpu.sync_copy(dst_hbm.at[wid], dst_v)
        plsc.subcore_barrier()

        def body(j, _):
            pltpu.sync_copy(ones_v, acc_sh.at[dst_v.at[j]], add=True)
            return 0

        lax.fori_loop(0, kb, body, 0)
        plsc.subcore_barrier()
        for r in range(ROWS_PER_TILE // EB):
            off = base + r * EB
            pltpu.sync_copy(acc_sh.at[pl.ds(off, EB)], tmp_v)
            pltpu.sync_copy(tmp_v, out_hbm.at[cid].at[pl.ds(off, EB)])

    return deg_kernel


def _make_agg_kernel(kb, do_c):
    mesh = plsc.VectorSubcoreMesh(core_axis_name="c", subcore_axis_name="s", num_cores=NC, num_subcores=NS)
    out_type = [jax.ShapeDtypeStruct((NC, NPAD, H), jnp.float32)]
    scratch = [
        pltpu.VMEM_SHARED((NPAD, H), jnp.float32),
        pltpu.VMEM((kb, EB), jnp.int32),
        pltpu.VMEM((kb, EB), jnp.int32),
        pltpu.VMEM((EB, H), jnp.float32),
        pltpu.SemaphoreType.DMA,
    ]
    if do_c:
        out_type.append(jax.ShapeDtypeStruct((NC, NPAD), jnp.float32))
        scratch += [
            pltpu.VMEM_SHARED((NPAD,), jnp.float32),
            pltpu.VMEM((NPAD,), jnp.float32),
            pltpu.VMEM((EB,), jnp.float32),
        ]

    def body_fn(*refs):
        if do_c:
            (y_hbm, src_hbm, dst_hbm, dinv_hbm, p_hbm, c_hbm,
             acc_sh, src_v, dst_v, rows_v, sem, c_sh, dinv_v, val_v) = refs
        else:
            (y_hbm, src_hbm, dst_hbm, p_hbm,
             acc_sh, src_v, dst_v, rows_v, sem) = refs
        cid = lax.axis_index("c")
        sid = lax.axis_index("s")
        wid = cid * NS + sid
        base = sid * ROWS_PER_TILE
        _zero_f32_rows(rows_v, EB)
        for r in range(ROWS_PER_TILE // EB):
            pltpu.sync_copy(rows_v, acc_sh.at[pl.ds(base + r * EB, EB)])
        if do_c:
            _zero_f32_vec(val_v, EB)
            for r in range(ROWS_PER_TILE // EB):
                pltpu.sync_copy(val_v, c_sh.at[pl.ds(base + r * EB, EB)])
            pltpu.sync_copy(dinv_hbm, dinv_v)
        pltpu.sync_copy(src_hbm.at[wid], src_v)
        pltpu.sync_copy(dst_hbm.at[wid], dst_v)
        plsc.subcore_barrier()

        def body(j, _):
            pltpu.async_copy(y_hbm.at[src_v.at[j]], rows_v, sem).wait()
            pltpu.sync_copy(rows_v, acc_sh.at[dst_v.at[j]], add=True)
            if do_c:
                for g in range(EB // 16):
                    didx = dst_v[j, pl.ds(g * 16, 16)]
                    val_v[pl.ds(g * 16, 16)] = plsc.load_gather(dinv_v, [didx])
                pltpu.sync_copy(val_v, c_sh.at[src_v.at[j]], add=True)
            return 0

        lax.fori_loop(0, kb, body, 0)
        plsc.subcore_barrier()
        for r in range(ROWS_PER_TILE // EB):
            off = base + r * EB
            pltpu.sync_copy(acc_sh.at[pl.ds(off, EB)], rows_v)
            pltpu.sync_copy(rows_v, p_hbm.at[cid].at[pl.ds(off, EB)])
        if do_c:
            for r in range(ROWS_PER_TILE // EB):
                off = base + r * EB
                pltpu.sync_copy(c_sh.at[pl.ds(off, EB)], val_v)
                pltpu.sync_copy(val_v, c_hbm.at[cid].at[pl.ds(off, EB)])

    return pl.kernel(
        body_fn, mesh=mesh, out_type=out_type, scratch_types=scratch,
        compiler_params=pltpu.CompilerParams(
            needs_layout_passes=False, use_tc_tiling_on_sc=False))


def _tc_k1(degp_ref, x_ref, w1_ref, dinv_ref, y1_ref):
    iota = lax.broadcasted_iota(jnp.int32, (NPAD, 1), 0)
    valid = iota < N_NODES
    deg = degp_ref[0] + degp_ref[1] + 1.0
    dinv = jnp.where(valid, lax.rsqrt(deg), 0.0)
    dinv_ref[...] = dinv
    xw = jnp.dot(x_ref[...], w1_ref[...], preferred_element_type=jnp.float32)
    y1_ref[...] = dinv * xw


def _tc_k2(p_ref, y_ref, dinv_ref, b_ref, w_ref, out_ref):
    dinv = dinv_ref[...]
    h = jax.nn.relu(dinv * (p_ref[0] + p_ref[1] + y_ref[...]) + b_ref[...])
    out_ref[...] = dinv * jnp.dot(h, w_ref[...],
                                  preferred_element_type=jnp.float32)


def _tc_k3(p_ref, y_ref, dinv_ref, b2_ref, cp_ref, w3_ref, b3_ref,
           lw_ref, lb_ref, out_ref):
    dinv = dinv_ref[...]
    h2 = jax.nn.relu(dinv * (p_ref[0] + p_ref[1] + y_ref[...]) + b2_ref[...])
    c = dinv * (cp_ref[0] + cp_ref[1] + dinv)
    ws = jnp.sum(c * h2, axis=0, keepdims=True)
    pooled = jnp.dot(ws, w3_ref[...],
                     preferred_element_type=jnp.float32) * (1.0 / N_NODES)
    pooled = pooled + b3_ref[...]
    logits = jnp.dot(pooled, lw_ref[...],
                     preferred_element_type=jnp.float32) + lb_ref[...]
    out_ref[...] = jax.nn.softmax(logits, axis=1)


def kernel(x, edge_index, W1, b1, W2, b2, W3, b3, lin_W, lin_b):
    n_edges = edge_index.shape[1]
    epad = -(-n_edges // (NW * EB)) * (NW * EB)
    kb = epad // (NW * EB)

    src = edge_index[0].astype(jnp.int32)
    dst = edge_index[1].astype(jnp.int32)
    pad_cfg = ((0, epad - n_edges),)
    srcr = jnp.pad(src, pad_cfg, constant_values=N_NODES).reshape(NW, kb, EB)
    dstr = jnp.pad(dst, pad_cfg, constant_values=N_NODES).reshape(NW, kb, EB)
    xpad = jnp.pad(x, ((0, NPAD - N_NODES), (0, 0)))

    degp = _make_deg_kernel(kb)(dstr)

    dinv_c, y1 = pl.pallas_call(
        _tc_k1,
        out_shape=(
            jax.ShapeDtypeStruct((NPAD, 1), jnp.float32),
            jax.ShapeDtypeStruct((NPAD, H), jnp.float32),
        ),
    )(degp.reshape(NC, NPAD, 1), xpad, W1)
    dinv_flat = dinv_c.reshape(NPAD)

    p1, cpart = _make_agg_kernel(kb, do_c=True)(y1, srcr, dstr, dinv_flat)

    y2 = pl.pallas_call(
        _tc_k2,
        out_shape=jax.ShapeDtypeStruct((NPAD, H), jnp.float32),
    )(p1, y1, dinv_c, b1.reshape(1, H), W2)

    (p2,) = _make_agg_kernel(kb, do_c=False)(y2, srcr, dstr)

    out = pl.pallas_call(
        _tc_k3,
        out_shape=jax.ShapeDtypeStruct((1, lin_W.shape[1]), jnp.float32),
    )(p2, y2, dinv_c, b2.reshape(1, H), cpart.reshape(NC, NPAD, 1), W3,
      b3.reshape(1, H), lin_W, lin_b.reshape(1, lin_W.shape[1]))
    return out


# speedup vs baseline: 36.3430x; 1.0740x over previous
import functools

import jax
import jax.numpy as jnp
from jax import lax
from jax.experimental import pallas as pl
from jax.experimental.pallas import tpu as pltpu
from jax.experimental.pallas import tpu_sc as plsc

N_NODES = 10000
H = 32
NC = 2
NS = 16
NW = NC * NS
EB = 128
NPAD = 10240
ROWS_PER_TILE = NPAD // NS


def _zero_f32_vec(ref, n):
    z = jnp.zeros((16,), jnp.float32)

    def body(i, _):
        ref[pl.ds(i * 16, 16)] = z
        return 0

    lax.fori_loop(0, n // 16, body, 0)


def _zero_f32_rows(ref, rows):
    z = jnp.zeros((16,), jnp.float32)

    def body(r, _):
        ref[r, pl.ds(0, 16)] = z
        ref[r, pl.ds(16, 16)] = z
        return 0

    lax.fori_loop(0, rows, body, 0)


def _make_deg_kernel(kb):
    mesh = plsc.VectorSubcoreMesh(core_axis_name="c", subcore_axis_name="s", num_cores=NC, num_subcores=NS)

    @functools.partial(
        pl.kernel,
        mesh=mesh,
        out_type=jax.ShapeDtypeStruct((NC, NPAD), jnp.float32),
        scratch_types=[
            pltpu.VMEM_SHARED((NPAD,), jnp.float32),
            pltpu.VMEM((kb, EB), jnp.int32),
            pltpu.VMEM((EB,), jnp.float32),
            pltpu.VMEM((EB,), jnp.float32),
        ],
    )
    def deg_kernel(dst_hbm, out_hbm, acc_sh, dst_v, ones_v, tmp_v):
        cid = lax.axis_index("c")
        sid = lax.axis_index("s")
        wid = cid * NS + sid
        one = jnp.ones((16,), jnp.float32)
        for g in range(EB // 16):
            ones_v[pl.ds(g * 16, 16)] = one
        _zero_f32_vec(tmp_v, EB)
        base = sid * ROWS_PER_TILE
        for r in range(ROWS_PER_TILE // EB):
            pltpu.sync_copy(tmp_v, acc_sh.at[pl.ds(base + r * EB, EB)])
        pltpu.sync_copy(dst_hbm.at[wid], dst_v)
        plsc.subcore_barrier()

        def body(j, _):
            pltpu.sync_copy(ones_v, acc_sh.at[dst_v.at[j]], add=True)
            return 0

        lax.fori_loop(0, kb, body, 0)
        plsc.subcore_barrier()
        for r in range(ROWS_PER_TILE // EB):
            off = base + r * EB
            pltpu.sync_copy(acc_sh.at[pl.ds(off, EB)], tmp_v)
            pltpu.sync_copy(tmp_v, out_hbm.at[cid].at[pl.ds(off, EB)])

    return deg_kernel


NBUF = 4


def _make_agg_kernel(kb, do_c):
    mesh = plsc.VectorSubcoreMesh(core_axis_name="c", subcore_axis_name="s", num_cores=NC, num_subcores=NS)
    out_type = [jax.ShapeDtypeStruct((NC, NPAD, H), jnp.float32)]
    scratch = [
        pltpu.VMEM_SHARED((NPAD, H), jnp.float32),
        pltpu.VMEM((kb, EB), jnp.int32),
        pltpu.VMEM((kb, EB), jnp.int32),
        [pltpu.VMEM((EB, H), jnp.float32)] * NBUF,
        [pltpu.SemaphoreType.DMA] * NBUF,
    ]
    if do_c:
        out_type.append(jax.ShapeDtypeStruct((NC, NPAD), jnp.float32))
        scratch += [
            pltpu.VMEM_SHARED((NPAD,), jnp.float32),
            pltpu.VMEM((NPAD,), jnp.float32),
            pltpu.VMEM((EB,), jnp.float32),
        ]

    def body_fn(*refs):
        if do_c:
            (y_hbm, src_hbm, dst_hbm, dinv_hbm, p_hbm, c_hbm,
             acc_sh, src_v, dst_v, rows, sems, c_sh, dinv_v, val_v) = refs
        else:
            (y_hbm, src_hbm, dst_hbm, p_hbm,
             acc_sh, src_v, dst_v, rows, sems) = refs
        cid = lax.axis_index("c")
        sid = lax.axis_index("s")
        wid = cid * NS + sid
        base = sid * ROWS_PER_TILE
        _zero_f32_rows(rows[0], EB)
        for r in range(ROWS_PER_TILE // EB):
            pltpu.sync_copy(rows[0], acc_sh.at[pl.ds(base + r * EB, EB)])
        if do_c:
            _zero_f32_vec(val_v, EB)
            for r in range(ROWS_PER_TILE // EB):
                pltpu.sync_copy(val_v, c_sh.at[pl.ds(base + r * EB, EB)])
            pltpu.sync_copy(dinv_hbm, dinv_v)
        pltpu.sync_copy(src_hbm.at[wid], src_v)
        pltpu.sync_copy(dst_hbm.at[wid], dst_v)
        plsc.subcore_barrier()

        for b in range(NBUF):
            pltpu.async_copy(y_hbm.at[src_v.at[b]], rows[b], sems[b])

        def body(g, _):
            j0 = g * NBUF
            for b in range(NBUF):
                j = j0 + b
                pltpu.make_async_copy(
                    y_hbm.at[src_v.at[j]], rows[b], sems[b]).wait()
                pltpu.sync_copy(rows[b], acc_sh.at[dst_v.at[j]], add=True)
                if do_c:
                    for gg in range(EB // 16):
                        didx = dst_v[j, pl.ds(gg * 16, 16)]
                        val_v[pl.ds(gg * 16, 16)] = plsc.load_gather(
                            dinv_v, [didx])
                    pltpu.sync_copy(val_v, c_sh.at[src_v.at[j]], add=True)
                nxt = j + NBUF

                @pl.when(nxt < kb)
                def _():
                    pltpu.async_copy(y_hbm.at[src_v.at[nxt]], rows[b],
                                     sems[b])
            return 0

        lax.fori_loop(0, kb // NBUF, body, 0)
        plsc.subcore_barrier()
        for r in range(ROWS_PER_TILE // EB):
            off = base + r * EB
            pltpu.sync_copy(acc_sh.at[pl.ds(off, EB)], rows[0])
            pltpu.sync_copy(rows[0], p_hbm.at[cid].at[pl.ds(off, EB)])
        if do_c:
            for r in range(ROWS_PER_TILE // EB):
                off = base + r * EB
                pltpu.sync_copy(c_sh.at[pl.ds(off, EB)], val_v)
                pltpu.sync_copy(val_v, c_hbm.at[cid].at[pl.ds(off, EB)])

    return pl.kernel(
        body_fn, mesh=mesh, out_type=out_type, scratch_types=scratch,
        compiler_params=pltpu.CompilerParams(
            needs_layout_passes=False, use_tc_tiling_on_sc=False))


def _tc_k1(degp_ref, x_ref, w1_ref, dinv_ref, y1_ref):
    iota = lax.broadcasted_iota(jnp.int32, (NPAD, 1), 0)
    valid = iota < N_NODES
    deg = degp_ref[0] + degp_ref[1] + 1.0
    dinv = jnp.where(valid, lax.rsqrt(deg), 0.0)
    dinv_ref[...] = dinv
    xw = jnp.dot(x_ref[...], w1_ref[...], preferred_element_type=jnp.float32)
    y1_ref[...] = dinv * xw


def _tc_k2(p_ref, y_ref, dinv_ref, b_ref, w_ref, out_ref):
    dinv = dinv_ref[...]
    h = jax.nn.relu(dinv * (p_ref[0] + p_ref[1] + y_ref[...]) + b_ref[...])
    out_ref[...] = dinv * jnp.dot(h, w_ref[...],
                                  preferred_element_type=jnp.float32)


def _tc_k3(p_ref, y_ref, dinv_ref, b2_ref, cp_ref, w3_ref, b3_ref,
           lw_ref, lb_ref, out_ref):
    dinv = dinv_ref[...]
    h2 = jax.nn.relu(dinv * (p_ref[0] + p_ref[1] + y_ref[...]) + b2_ref[...])
    c = dinv * (cp_ref[0] + cp_ref[1] + dinv)
    ws = jnp.sum(c * h2, axis=0, keepdims=True)
    pooled = jnp.dot(ws, w3_ref[...],
                     preferred_element_type=jnp.float32) * (1.0 / N_NODES)
    pooled = pooled + b3_ref[...]
    logits = jnp.dot(pooled, lw_ref[...],
                     preferred_element_type=jnp.float32) + lb_ref[...]
    out_ref[...] = jax.nn.softmax(logits, axis=1)


def kernel(x, edge_index, W1, b1, W2, b2, W3, b3, lin_W, lin_b):
    n_edges = edge_index.shape[1]
    chunk = NW * EB * NBUF
    epad = -(-n_edges // chunk) * chunk
    kb = epad // (NW * EB)

    src = edge_index[0].astype(jnp.int32)
    dst = edge_index[1].astype(jnp.int32)
    pad_cfg = ((0, epad - n_edges),)
    srcr = jnp.pad(src, pad_cfg, constant_values=N_NODES).reshape(NW, kb, EB)
    dstr = jnp.pad(dst, pad_cfg, constant_values=N_NODES).reshape(NW, kb, EB)
    xpad = jnp.pad(x, ((0, NPAD - N_NODES), (0, 0)))

    degp = _make_deg_kernel(kb)(dstr)

    dinv_c, y1 = pl.pallas_call(
        _tc_k1,
        out_shape=(
            jax.ShapeDtypeStruct((NPAD, 1), jnp.float32),
            jax.ShapeDtypeStruct((NPAD, H), jnp.float32),
        ),
    )(degp.reshape(NC, NPAD, 1), xpad, W1)
    dinv_flat = dinv_c.reshape(NPAD)

    p1, cpart = _make_agg_kernel(kb, do_c=True)(y1, srcr, dstr, dinv_flat)

    y2 = pl.pallas_call(
        _tc_k2,
        out_shape=jax.ShapeDtypeStruct((NPAD, H), jnp.float32),
    )(p1, y1, dinv_c, b1.reshape(1, H), W2)

    (p2,) = _make_agg_kernel(kb, do_c=False)(y2, srcr, dstr)

    out = pl.pallas_call(
        _tc_k3,
        out_shape=jax.ShapeDtypeStruct((1, lin_W.shape[1]), jnp.float32),
    )(p2, y2, dinv_c, b2.reshape(1, H), cpart.reshape(NC, NPAD, 1), W3,
      b3.reshape(1, H), lin_W, lin_b.reshape(1, lin_W.shape[1]))
    return out


# speedup vs baseline: 36.8776x; 1.0147x over previous
import functools

import jax
import jax.numpy as jnp
from jax import lax
from jax.experimental import pallas as pl
from jax.experimental.pallas import tpu as pltpu
from jax.experimental.pallas import tpu_sc as plsc

N_NODES = 10000
H = 32
NC = 2
NS = 16
NW = NC * NS
EB = 128
NPAD = 10240
ROWS_PER_TILE = NPAD // NS


def _zero_f32_vec(ref, n):
    z = jnp.zeros((16,), jnp.float32)

    def body(i, _):
        ref[pl.ds(i * 16, 16)] = z
        return 0

    lax.fori_loop(0, n // 16, body, 0)


def _zero_f32_rows(ref, rows):
    z = jnp.zeros((16,), jnp.float32)

    def body(r, _):
        ref[r, pl.ds(0, 16)] = z
        ref[r, pl.ds(16, 16)] = z
        return 0

    lax.fori_loop(0, rows, body, 0)


def _make_deg_kernel(kb):
    mesh = plsc.VectorSubcoreMesh(core_axis_name="c", subcore_axis_name="s", num_cores=NC, num_subcores=NS)

    @functools.partial(
        pl.kernel,
        mesh=mesh,
        out_type=jax.ShapeDtypeStruct((NC, NPAD), jnp.float32),
        scratch_types=[
            pltpu.VMEM_SHARED((NPAD,), jnp.float32),
            pltpu.VMEM((kb, EB), jnp.int32),
            pltpu.VMEM((EB,), jnp.float32),
            pltpu.VMEM((EB,), jnp.float32),
        ],
    )
    def deg_kernel(dst_hbm, out_hbm, acc_sh, dst_v, ones_v, tmp_v):
        cid = lax.axis_index("c")
        sid = lax.axis_index("s")
        wid = cid * NS + sid
        one = jnp.ones((16,), jnp.float32)
        for g in range(EB // 16):
            ones_v[pl.ds(g * 16, 16)] = one
        _zero_f32_vec(tmp_v, EB)
        base = sid * ROWS_PER_TILE
        for r in range(ROWS_PER_TILE // EB):
            pltpu.sync_copy(tmp_v, acc_sh.at[pl.ds(base + r * EB, EB)])
        pltpu.sync_copy(dst_hbm.at[wid], dst_v)
        plsc.subcore_barrier()

        def body(j, _):
            pltpu.sync_copy(ones_v, acc_sh.at[dst_v.at[j]], add=True)
            return 0

        lax.fori_loop(0, kb, body, 0)
        plsc.subcore_barrier()
        for r in range(ROWS_PER_TILE // EB):
            off = base + r * EB
            pltpu.sync_copy(acc_sh.at[pl.ds(off, EB)], tmp_v)
            pltpu.sync_copy(tmp_v, out_hbm.at[cid].at[pl.ds(off, EB)])

    return deg_kernel


NBUF = 8


def _make_agg_kernel(kb, do_c):
    mesh = plsc.VectorSubcoreMesh(core_axis_name="c", subcore_axis_name="s", num_cores=NC, num_subcores=NS)
    out_type = [jax.ShapeDtypeStruct((NC, NPAD, H), jnp.float32)]
    scratch = [
        pltpu.VMEM_SHARED((NPAD, H), jnp.float32),
        pltpu.VMEM((kb, EB), jnp.int32),
        pltpu.VMEM((kb, EB), jnp.int32),
        [pltpu.VMEM((EB, H), jnp.float32)] * NBUF,
        [pltpu.SemaphoreType.DMA] * NBUF,
        [pltpu.SemaphoreType.DMA] * NBUF,
    ]
    if do_c:
        out_type.append(jax.ShapeDtypeStruct((NC, NPAD), jnp.float32))
        scratch += [
            pltpu.VMEM_SHARED((NPAD,), jnp.float32),
            pltpu.VMEM((NPAD,), jnp.float32),
            [pltpu.VMEM((EB,), jnp.float32)] * NBUF,
            [pltpu.SemaphoreType.DMA] * NBUF,
        ]

    def body_fn(*refs):
        if do_c:
            (y_hbm, src_hbm, dst_hbm, dinv_hbm, p_hbm, c_hbm,
             acc_sh, src_v, dst_v, rows, gsems, ssems,
             c_sh, dinv_v, vals, vsems) = refs
        else:
            (y_hbm, src_hbm, dst_hbm, p_hbm,
             acc_sh, src_v, dst_v, rows, gsems, ssems) = refs
        cid = lax.axis_index("c")
        sid = lax.axis_index("s")
        wid = cid * NS + sid
        base = sid * ROWS_PER_TILE
        _zero_f32_rows(rows[0], EB)
        for r in range(ROWS_PER_TILE // EB):
            pltpu.sync_copy(rows[0], acc_sh.at[pl.ds(base + r * EB, EB)])
        if do_c:
            _zero_f32_vec(vals[0], EB)
            for r in range(ROWS_PER_TILE // EB):
                pltpu.sync_copy(vals[0], c_sh.at[pl.ds(base + r * EB, EB)])
            pltpu.sync_copy(dinv_hbm, dinv_v)
        pltpu.sync_copy(src_hbm.at[wid], src_v)
        pltpu.sync_copy(dst_hbm.at[wid], dst_v)
        plsc.subcore_barrier()

        for b in range(NBUF):
            pltpu.async_copy(y_hbm.at[src_v.at[b]], rows[b], gsems[b])

        def body(g, _):
            j0 = g * NBUF
            for b in range(NBUF):
                j = j0 + b
                pltpu.make_async_copy(
                    y_hbm.at[src_v.at[j]], rows[b], gsems[b]).wait()
                pltpu.async_copy(rows[b], acc_sh.at[dst_v.at[j]], ssems[b],
                                 add=True)
                if do_c:
                    @pl.when(j >= NBUF)
                    def _():
                        pltpu.make_async_copy(
                            vals[b], c_sh.at[src_v.at[j]], vsems[b]).wait()

                    for gg in range(EB // 16):
                        didx = dst_v[j, pl.ds(gg * 16, 16)]
                        vals[b][pl.ds(gg * 16, 16)] = plsc.load_gather(
                            dinv_v, [didx])
                    pltpu.async_copy(vals[b], c_sh.at[src_v.at[j]], vsems[b],
                                     add=True)
                nxt = j + NBUF

                @pl.when(nxt < kb)
                def _():
                    pltpu.make_async_copy(
                        rows[b], acc_sh.at[dst_v.at[j]], ssems[b]).wait()
                    pltpu.async_copy(y_hbm.at[src_v.at[nxt]], rows[b],
                                     gsems[b])
            return 0

        lax.fori_loop(0, kb // NBUF, body, 0)
        last = kb - NBUF
        for b in range(NBUF):
            pltpu.make_async_copy(
                rows[b], acc_sh.at[dst_v.at[last + b]], ssems[b]).wait()
            if do_c:
                pltpu.make_async_copy(
                    vals[b], c_sh.at[src_v.at[last + b]], vsems[b]).wait()
        plsc.subcore_barrier()
        for r in range(ROWS_PER_TILE // EB):
            off = base + r * EB
            pltpu.sync_copy(acc_sh.at[pl.ds(off, EB)], rows[0])
            pltpu.sync_copy(rows[0], p_hbm.at[cid].at[pl.ds(off, EB)])
        if do_c:
            for r in range(ROWS_PER_TILE // EB):
                off = base + r * EB
                pltpu.sync_copy(c_sh.at[pl.ds(off, EB)], vals[0])
                pltpu.sync_copy(vals[0], c_hbm.at[cid].at[pl.ds(off, EB)])

    return pl.kernel(
        body_fn, mesh=mesh, out_type=out_type, scratch_types=scratch,
        compiler_params=pltpu.CompilerParams(
            needs_layout_passes=False, use_tc_tiling_on_sc=False))


def _tc_k1(degp_ref, x_ref, w1_ref, dinv_ref, y1_ref):
    iota = lax.broadcasted_iota(jnp.int32, (NPAD, 1), 0)
    valid = iota < N_NODES
    deg = degp_ref[0] + degp_ref[1] + 1.0
    dinv = jnp.where(valid, lax.rsqrt(deg), 0.0)
    dinv_ref[...] = dinv
    xw = jnp.dot(x_ref[...], w1_ref[...], preferred_element_type=jnp.float32)
    y1_ref[...] = dinv * xw


def _tc_k2(p_ref, y_ref, dinv_ref, b_ref, w_ref, out_ref):
    dinv = dinv_ref[...]
    h = jax.nn.relu(dinv * (p_ref[0] + p_ref[1] + y_ref[...]) + b_ref[...])
    out_ref[...] = dinv * jnp.dot(h, w_ref[...],
                                  preferred_element_type=jnp.float32)


def _tc_k3(p_ref, y_ref, dinv_ref, b2_ref, cp_ref, w3_ref, b3_ref,
           lw_ref, lb_ref, out_ref):
    dinv = dinv_ref[...]
    h2 = jax.nn.relu(dinv * (p_ref[0] + p_ref[1] + y_ref[...]) + b2_ref[...])
    c = dinv * (cp_ref[0] + cp_ref[1] + dinv)
    ws = jnp.sum(c * h2, axis=0, keepdims=True)
    pooled = jnp.dot(ws, w3_ref[...],
                     preferred_element_type=jnp.float32) * (1.0 / N_NODES)
    pooled = pooled + b3_ref[...]
    logits = jnp.dot(pooled, lw_ref[...],
                     preferred_element_type=jnp.float32) + lb_ref[...]
    out_ref[...] = jax.nn.softmax(logits, axis=1)


def kernel(x, edge_index, W1, b1, W2, b2, W3, b3, lin_W, lin_b):
    n_edges = edge_index.shape[1]
    chunk = NW * EB * NBUF
    epad = -(-n_edges // chunk) * chunk
    kb = epad // (NW * EB)

    src = edge_index[0].astype(jnp.int32)
    dst = edge_index[1].astype(jnp.int32)
    pad_cfg = ((0, epad - n_edges),)
    srcr = jnp.pad(src, pad_cfg, constant_values=N_NODES).reshape(NW, kb, EB)
    dstr = jnp.pad(dst, pad_cfg, constant_values=N_NODES).reshape(NW, kb, EB)
    xpad = jnp.pad(x, ((0, NPAD - N_NODES), (0, 0)))

    degp = _make_deg_kernel(kb)(dstr)

    dinv_c, y1 = pl.pallas_call(
        _tc_k1,
        out_shape=(
            jax.ShapeDtypeStruct((NPAD, 1), jnp.float32),
            jax.ShapeDtypeStruct((NPAD, H), jnp.float32),
        ),
    )(degp.reshape(NC, NPAD, 1), xpad, W1)
    dinv_flat = dinv_c.reshape(NPAD)

    p1, cpart = _make_agg_kernel(kb, do_c=True)(y1, srcr, dstr, dinv_flat)

    y2 = pl.pallas_call(
        _tc_k2,
        out_shape=jax.ShapeDtypeStruct((NPAD, H), jnp.float32),
    )(p1, y1, dinv_c, b1.reshape(1, H), W2)

    (p2,) = _make_agg_kernel(kb, do_c=False)(y2, srcr, dstr)

    out = pl.pallas_call(
        _tc_k3,
        out_shape=jax.ShapeDtypeStruct((1, lin_W.shape[1]), jnp.float32),
    )(p2, y2, dinv_c, b2.reshape(1, H), cpart.reshape(NC, NPAD, 1), W3,
      b3.reshape(1, H), lin_W, lin_b.reshape(1, lin_W.shape[1]))
    return out


# speedup vs baseline: 55.0691x; 1.4933x over previous
import functools

import jax
import jax.numpy as jnp
from jax import lax
from jax.experimental import pallas as pl
from jax.experimental.pallas import tpu as pltpu
from jax.experimental.pallas import tpu_sc as plsc

N_NODES = 10000
H = 32
NC = 2
NS = 16
NW = NC * NS
EB = 128
NPAD = 10240
ROWS_PER_TILE = NPAD // NS


def _zero_f32_vec(ref, n):
    z = jnp.zeros((16,), jnp.float32)

    def body(i, _):
        ref[pl.ds(i * 16, 16)] = z
        return 0

    lax.fori_loop(0, n // 16, body, 0)


def _zero_f32_rows(ref, rows):
    z = jnp.zeros((16,), jnp.float32)

    def body(r, _):
        ref[r, pl.ds(0, 16)] = z
        ref[r, pl.ds(16, 16)] = z
        return 0

    lax.fori_loop(0, rows, body, 0)


def _make_deg_kernel(kb):
    mesh = plsc.VectorSubcoreMesh(core_axis_name="c", subcore_axis_name="s", num_cores=NC, num_subcores=NS)

    @functools.partial(
        pl.kernel,
        mesh=mesh,
        out_type=jax.ShapeDtypeStruct((NC, NPAD), jnp.float32),
        scratch_types=[
            pltpu.VMEM_SHARED((NPAD,), jnp.float32),
            pltpu.VMEM((kb, EB), jnp.int32),
            pltpu.VMEM((EB,), jnp.float32),
            pltpu.VMEM((EB,), jnp.float32),
        ],
    )
    def deg_kernel(dst_hbm, out_hbm, acc_sh, dst_v, ones_v, tmp_v):
        cid = lax.axis_index("c")
        sid = lax.axis_index("s")
        wid = cid * NS + sid
        one = jnp.ones((16,), jnp.float32)
        for g in range(EB // 16):
            ones_v[pl.ds(g * 16, 16)] = one
        _zero_f32_vec(tmp_v, EB)
        base = sid * ROWS_PER_TILE
        for r in range(ROWS_PER_TILE // EB):
            pltpu.sync_copy(tmp_v, acc_sh.at[pl.ds(base + r * EB, EB)])
        pltpu.sync_copy(dst_hbm.at[wid], dst_v)
        plsc.subcore_barrier()

        def body(j, _):
            pltpu.sync_copy(ones_v, acc_sh.at[dst_v.at[j]], add=True)
            return 0

        lax.fori_loop(0, kb, body, 0)
        plsc.subcore_barrier()
        for r in range(ROWS_PER_TILE // EB):
            off = base + r * EB
            pltpu.sync_copy(acc_sh.at[pl.ds(off, EB)], tmp_v)
            pltpu.sync_copy(tmp_v, out_hbm.at[cid].at[pl.ds(off, EB)])

    return deg_kernel


NBUF = 8


def _make_agg_kernel(kb, do_c):
    mesh = plsc.VectorSubcoreMesh(core_axis_name="c", subcore_axis_name="s", num_cores=NC, num_subcores=NS)
    out_type = [jax.ShapeDtypeStruct((NC, NPAD, H), jnp.float32)]
    scratch = [
        pltpu.VMEM_SHARED((NPAD, H), jnp.float32),
        pltpu.VMEM_SHARED((NPAD, H), jnp.float32),
        pltpu.VMEM((kb, EB), jnp.int32),
        pltpu.VMEM((kb, EB), jnp.int32),
        [pltpu.VMEM((EB, H), jnp.float32)] * NBUF,
        [pltpu.SemaphoreType.DMA] * NBUF,
        [pltpu.SemaphoreType.DMA] * NBUF,
    ]
    if do_c:
        out_type.append(jax.ShapeDtypeStruct((NC, NPAD), jnp.float32))
        scratch += [
            pltpu.VMEM_SHARED((NPAD,), jnp.float32),
            pltpu.VMEM((NPAD,), jnp.float32),
            [pltpu.VMEM((EB,), jnp.float32)] * NBUF,
            [pltpu.SemaphoreType.DMA] * NBUF,
        ]

    def body_fn(*refs):
        if do_c:
            (y_hbm, src_hbm, dst_hbm, dinv_hbm, p_hbm, c_hbm,
             acc_sh, y_sh, src_v, dst_v, rows, gsems, ssems,
             c_sh, dinv_v, vals, vsems) = refs
        else:
            (y_hbm, src_hbm, dst_hbm, p_hbm,
             acc_sh, y_sh, src_v, dst_v, rows, gsems, ssems) = refs
        cid = lax.axis_index("c")
        sid = lax.axis_index("s")
        wid = cid * NS + sid
        base = sid * ROWS_PER_TILE
        _zero_f32_rows(rows[0], EB)
        for r in range(ROWS_PER_TILE // EB):
            pltpu.sync_copy(rows[0], acc_sh.at[pl.ds(base + r * EB, EB)])
        for r in range(ROWS_PER_TILE // EB):
            off = base + r * EB
            pltpu.sync_copy(y_hbm.at[pl.ds(off, EB)], rows[1])
            pltpu.sync_copy(rows[1], y_sh.at[pl.ds(off, EB)])
        if do_c:
            _zero_f32_vec(vals[0], EB)
            for r in range(ROWS_PER_TILE // EB):
                pltpu.sync_copy(vals[0], c_sh.at[pl.ds(base + r * EB, EB)])
            pltpu.sync_copy(dinv_hbm, dinv_v)
        pltpu.sync_copy(src_hbm.at[wid], src_v)
        pltpu.sync_copy(dst_hbm.at[wid], dst_v)
        plsc.subcore_barrier()

        for b in range(NBUF):
            pltpu.async_copy(y_sh.at[src_v.at[b]], rows[b], gsems[b])

        def body(g, _):
            j0 = g * NBUF
            for b in range(NBUF):
                j = j0 + b
                pltpu.make_async_copy(
                    y_sh.at[src_v.at[j]], rows[b], gsems[b]).wait()
                pltpu.async_copy(rows[b], acc_sh.at[dst_v.at[j]], ssems[b],
                                 add=True)
                if do_c:
                    @pl.when(j >= NBUF)
                    def _():
                        pltpu.make_async_copy(
                            vals[b], c_sh.at[src_v.at[j]], vsems[b]).wait()

                    for gg in range(EB // 16):
                        didx = dst_v[j, pl.ds(gg * 16, 16)]
                        vals[b][pl.ds(gg * 16, 16)] = plsc.load_gather(
                            dinv_v, [didx])
                    pltpu.async_copy(vals[b], c_sh.at[src_v.at[j]], vsems[b],
                                     add=True)
                nxt = j + NBUF

                @pl.when(nxt < kb)
                def _():
                    pltpu.make_async_copy(
                        rows[b], acc_sh.at[dst_v.at[j]], ssems[b]).wait()
                    pltpu.async_copy(y_sh.at[src_v.at[nxt]], rows[b],
                                     gsems[b])
            return 0

        lax.fori_loop(0, kb // NBUF, body, 0)
        last = kb - NBUF
        for b in range(NBUF):
            pltpu.make_async_copy(
                rows[b], acc_sh.at[dst_v.at[last + b]], ssems[b]).wait()
            if do_c:
                pltpu.make_async_copy(
                    vals[b], c_sh.at[src_v.at[last + b]], vsems[b]).wait()
        plsc.subcore_barrier()
        for r in range(ROWS_PER_TILE // EB):
            off = base + r * EB
            pltpu.sync_copy(acc_sh.at[pl.ds(off, EB)], rows[0])
            pltpu.sync_copy(rows[0], p_hbm.at[cid].at[pl.ds(off, EB)])
        if do_c:
            for r in range(ROWS_PER_TILE // EB):
                off = base + r * EB
                pltpu.sync_copy(c_sh.at[pl.ds(off, EB)], vals[0])
                pltpu.sync_copy(vals[0], c_hbm.at[cid].at[pl.ds(off, EB)])

    return pl.kernel(
        body_fn, mesh=mesh, out_type=out_type, scratch_types=scratch,
        compiler_params=pltpu.CompilerParams(
            needs_layout_passes=False, use_tc_tiling_on_sc=False))


def _tc_k1(degp_ref, x_ref, w1_ref, dinv_ref, y1_ref):
    iota = lax.broadcasted_iota(jnp.int32, (NPAD, 1), 0)
    valid = iota < N_NODES
    deg = degp_ref[0] + degp_ref[1] + 1.0
    dinv = jnp.where(valid, lax.rsqrt(deg), 0.0)
    dinv_ref[...] = dinv
    xw = jnp.dot(x_ref[...], w1_ref[...], preferred_element_type=jnp.float32)
    y1_ref[...] = dinv * xw


def _tc_k2(p_ref, y_ref, dinv_ref, b_ref, w_ref, out_ref):
    dinv = dinv_ref[...]
    h = jax.nn.relu(dinv * (p_ref[0] + p_ref[1] + y_ref[...]) + b_ref[...])
    out_ref[...] = dinv * jnp.dot(h, w_ref[...],
                                  preferred_element_type=jnp.float32)


def _tc_k3(p_ref, y_ref, dinv_ref, b2_ref, cp_ref, w3_ref, b3_ref,
           lw_ref, lb_ref, out_ref):
    dinv = dinv_ref[...]
    h2 = jax.nn.relu(dinv * (p_ref[0] + p_ref[1] + y_ref[...]) + b2_ref[...])
    c = dinv * (cp_ref[0] + cp_ref[1] + dinv)
    ws = jnp.sum(c * h2, axis=0, keepdims=True)
    pooled = jnp.dot(ws, w3_ref[...],
                     preferred_element_type=jnp.float32) * (1.0 / N_NODES)
    pooled = pooled + b3_ref[...]
    logits = jnp.dot(pooled, lw_ref[...],
                     preferred_element_type=jnp.float32) + lb_ref[...]
    out_ref[...] = jax.nn.softmax(logits, axis=1)


def kernel(x, edge_index, W1, b1, W2, b2, W3, b3, lin_W, lin_b):
    n_edges = edge_index.shape[1]
    chunk = NW * EB * NBUF
    epad = -(-n_edges // chunk) * chunk
    kb = epad // (NW * EB)

    src = edge_index[0].astype(jnp.int32)
    dst = edge_index[1].astype(jnp.int32)
    pad_cfg = ((0, epad - n_edges),)
    srcr = jnp.pad(src, pad_cfg, constant_values=N_NODES).reshape(NW, kb, EB)
    dstr = jnp.pad(dst, pad_cfg, constant_values=N_NODES).reshape(NW, kb, EB)
    xpad = jnp.pad(x, ((0, NPAD - N_NODES), (0, 0)))

    degp = _make_deg_kernel(kb)(dstr)

    dinv_c, y1 = pl.pallas_call(
        _tc_k1,
        out_shape=(
            jax.ShapeDtypeStruct((NPAD, 1), jnp.float32),
            jax.ShapeDtypeStruct((NPAD, H), jnp.float32),
        ),
    )(degp.reshape(NC, NPAD, 1), xpad, W1)
    dinv_flat = dinv_c.reshape(NPAD)

    p1, cpart = _make_agg_kernel(kb, do_c=True)(y1, srcr, dstr, dinv_flat)

    y2 = pl.pallas_call(
        _tc_k2,
        out_shape=jax.ShapeDtypeStruct((NPAD, H), jnp.float32),
    )(p1, y1, dinv_c, b1.reshape(1, H), W2)

    (p2,) = _make_agg_kernel(kb, do_c=False)(y2, srcr, dstr)

    out = pl.pallas_call(
        _tc_k3,
        out_shape=jax.ShapeDtypeStruct((1, lin_W.shape[1]), jnp.float32),
    )(p2, y2, dinv_c, b2.reshape(1, H), cpart.reshape(NC, NPAD, 1), W3,
      b3.reshape(1, H), lin_W, lin_b.reshape(1, lin_W.shape[1]))
    return out


# speedup vs baseline: 58.1081x; 1.0552x over previous
import functools

import jax
import jax.numpy as jnp
from jax import lax
from jax.experimental import pallas as pl
from jax.experimental.pallas import tpu as pltpu
from jax.experimental.pallas import tpu_sc as plsc

N_NODES = 10000
H = 32
NC = 2
NS = 16
NW = NC * NS
EB = 128
NPAD = 10240
ROWS_PER_TILE = NPAD // NS


def _zero_f32_vec(ref, n):
    z = jnp.zeros((16,), jnp.float32)

    def body(i, _):
        ref[pl.ds(i * 16, 16)] = z
        return 0

    lax.fori_loop(0, n // 16, body, 0)


def _zero_f32_rows(ref, rows):
    z = jnp.zeros((16,), jnp.float32)

    def body(r, _):
        ref[r, pl.ds(0, 16)] = z
        ref[r, pl.ds(16, 16)] = z
        return 0

    lax.fori_loop(0, rows, body, 0)


def _make_deg_kernel(kb):
    mesh = plsc.VectorSubcoreMesh(core_axis_name="c", subcore_axis_name="s", num_cores=NC, num_subcores=NS)

    @functools.partial(
        pl.kernel,
        mesh=mesh,
        out_type=jax.ShapeDtypeStruct((NC, NPAD), jnp.float32),
        scratch_types=[
            pltpu.VMEM_SHARED((NPAD,), jnp.float32),
            pltpu.VMEM((kb, EB), jnp.int32),
            pltpu.VMEM((EB,), jnp.float32),
            pltpu.VMEM((EB,), jnp.float32),
        ],
    )
    def deg_kernel(dst_hbm, out_hbm, acc_sh, dst_v, ones_v, tmp_v):
        cid = lax.axis_index("c")
        sid = lax.axis_index("s")
        wid = cid * NS + sid
        one = jnp.ones((16,), jnp.float32)
        for g in range(EB // 16):
            ones_v[pl.ds(g * 16, 16)] = one
        _zero_f32_vec(tmp_v, EB)
        base = sid * ROWS_PER_TILE
        for r in range(ROWS_PER_TILE // EB):
            pltpu.sync_copy(tmp_v, acc_sh.at[pl.ds(base + r * EB, EB)])
        pltpu.sync_copy(dst_hbm.at[wid], dst_v)
        plsc.subcore_barrier()

        def body(j, _):
            pltpu.sync_copy(ones_v, acc_sh.at[dst_v.at[j]], add=True)
            return 0

        lax.fori_loop(0, kb, body, 0)
        plsc.subcore_barrier()
        for r in range(ROWS_PER_TILE // EB):
            off = base + r * EB
            pltpu.sync_copy(acc_sh.at[pl.ds(off, EB)], tmp_v)
            pltpu.sync_copy(tmp_v, out_hbm.at[cid].at[pl.ds(off, EB)])

    return deg_kernel


NBUF = 8


def _make_agg_kernel(kb, do_c):
    mesh = plsc.VectorSubcoreMesh(core_axis_name="c", subcore_axis_name="s", num_cores=NC, num_subcores=NS)
    out_type = [jax.ShapeDtypeStruct((NC, NPAD, H), jnp.float32)]
    scratch = [
        pltpu.VMEM_SHARED((NPAD, H), jnp.float32),
        pltpu.VMEM_SHARED((NPAD, H), jnp.float32),
        pltpu.VMEM((kb, EB), jnp.int32),
        pltpu.VMEM((kb, EB), jnp.int32),
        [pltpu.VMEM((EB, H), jnp.float32)] * NBUF,
        [pltpu.SemaphoreType.DMA] * NBUF,
        [pltpu.SemaphoreType.DMA] * NBUF,
    ]
    if do_c:
        out_type.append(jax.ShapeDtypeStruct((NW, NPAD), jnp.float32))
        scratch += [
            pltpu.VMEM((NPAD,), jnp.float32),
            pltpu.VMEM((NPAD,), jnp.float32),
        ]

    nstg = ROWS_PER_TILE // EB

    def body_fn(*refs):
        if do_c:
            (y_hbm, src_hbm, dst_hbm, dinv_hbm, p_hbm, c_hbm,
             acc_sh, y_sh, src_v, dst_v, rows, gsems, ssems,
             dinv_v, cacc_v) = refs
        else:
            (y_hbm, src_hbm, dst_hbm, p_hbm,
             acc_sh, y_sh, src_v, dst_v, rows, gsems, ssems) = refs
        cid = lax.axis_index("c")
        sid = lax.axis_index("s")
        wid = cid * NS + sid
        base = sid * ROWS_PER_TILE
        _zero_f32_rows(rows[NBUF - 1], EB)
        for r in range(nstg):
            off = base + r * EB
            pltpu.async_copy(y_hbm.at[pl.ds(off, EB)], rows[r], gsems[r])
            pltpu.async_copy(rows[NBUF - 1], acc_sh.at[pl.ds(off, EB)],
                             ssems[r])
        pltpu.sync_copy(src_hbm.at[wid], src_v)
        pltpu.sync_copy(dst_hbm.at[wid], dst_v)
        if do_c:
            pltpu.sync_copy(dinv_hbm, dinv_v)
            _zero_f32_vec(cacc_v, NPAD)
        for r in range(nstg):
            off = base + r * EB
            pltpu.make_async_copy(
                y_hbm.at[pl.ds(off, EB)], rows[r], gsems[r]).wait()
            pltpu.async_copy(rows[r], y_sh.at[pl.ds(off, EB)], gsems[r])
        for r in range(nstg):
            off = base + r * EB
            pltpu.make_async_copy(
                rows[r], y_sh.at[pl.ds(off, EB)], gsems[r]).wait()
            pltpu.make_async_copy(
                rows[NBUF - 1], acc_sh.at[pl.ds(off, EB)], ssems[r]).wait()
        plsc.subcore_barrier()

        for b in range(NBUF):
            pltpu.async_copy(y_sh.at[src_v.at[b]], rows[b], gsems[b])

        def body(g, _):
            j0 = g * NBUF
            for b in range(NBUF):
                j = j0 + b
                pltpu.make_async_copy(
                    y_sh.at[src_v.at[j]], rows[b], gsems[b]).wait()
                pltpu.async_copy(rows[b], acc_sh.at[dst_v.at[j]], ssems[b],
                                 add=True)
                if do_c:
                    for gg in range(EB // 16):
                        didx = dst_v[j, pl.ds(gg * 16, 16)]
                        sidx = src_v[j, pl.ds(gg * 16, 16)]
                        vals16 = plsc.load_gather(dinv_v, [didx])
                        plsc.addupdate_scatter(cacc_v, [sidx], vals16)
                nxt = j + NBUF

                @pl.when(nxt < kb)
                def _():
                    pltpu.make_async_copy(
                        rows[b], acc_sh.at[dst_v.at[j]], ssems[b]).wait()
                    pltpu.async_copy(y_sh.at[src_v.at[nxt]], rows[b],
                                     gsems[b])
            return 0

        lax.fori_loop(0, kb // NBUF, body, 0)
        if do_c:
            pltpu.async_copy(cacc_v, c_hbm.at[wid], gsems[NBUF - 1])
        last = kb - NBUF
        for b in range(NBUF):
            pltpu.make_async_copy(
                rows[b], acc_sh.at[dst_v.at[last + b]], ssems[b]).wait()
        plsc.subcore_barrier()
        for r in range(nstg):
            off = base + r * EB
            pltpu.sync_copy(acc_sh.at[pl.ds(off, EB)], rows[0])
            pltpu.sync_copy(rows[0], p_hbm.at[cid].at[pl.ds(off, EB)])
        if do_c:
            pltpu.make_async_copy(cacc_v, c_hbm.at[wid], gsems[NBUF - 1]).wait()

    return pl.kernel(
        body_fn, mesh=mesh, out_type=out_type, scratch_types=scratch,
        compiler_params=pltpu.CompilerParams(
            needs_layout_passes=False, use_tc_tiling_on_sc=False))


RB = 1280


def _tc_k1(degp_ref, x_ref, w1_ref, dinv_ref, y1_ref):
    i = pl.program_id(0)
    iota = i * RB + lax.broadcasted_iota(jnp.int32, (RB, 1), 0)
    valid = iota < N_NODES
    deg = degp_ref[0] + degp_ref[1] + 1.0
    dinv = jnp.where(valid, lax.rsqrt(deg), 0.0)
    dinv_ref[...] = dinv
    xw = jnp.dot(x_ref[...], w1_ref[...], preferred_element_type=jnp.float32)
    y1_ref[...] = dinv * xw


def _tc_k2(p_ref, y_ref, dinv_ref, b_ref, w_ref, out_ref):
    dinv = dinv_ref[...]
    h = jax.nn.relu(dinv * (p_ref[0] + p_ref[1] + y_ref[...]) + b_ref[...])
    out_ref[...] = dinv * jnp.dot(h, w_ref[...],
                                  preferred_element_type=jnp.float32)


def _tc_k3(p_ref, y_ref, dinv_ref, b2_ref, cp_ref, w3_ref, b3_ref,
           lw_ref, lb_ref, out_ref):
    dinv = dinv_ref[...]
    h2 = jax.nn.relu(dinv * (p_ref[0] + p_ref[1] + y_ref[...]) + b2_ref[...])
    c = dinv * (jnp.sum(cp_ref[...], axis=1, keepdims=True) + dinv)
    ws = jnp.sum(c * h2, axis=0, keepdims=True)
    pooled = jnp.dot(ws, w3_ref[...],
                     preferred_element_type=jnp.float32) * (1.0 / N_NODES)
    pooled = pooled + b3_ref[...]
    logits = jnp.dot(pooled, lw_ref[...],
                     preferred_element_type=jnp.float32) + lb_ref[...]
    out_ref[...] = jax.nn.softmax(logits, axis=1)


def kernel(x, edge_index, W1, b1, W2, b2, W3, b3, lin_W, lin_b):
    n_edges = edge_index.shape[1]
    chunk = NW * EB * NBUF
    epad = -(-n_edges // chunk) * chunk
    kb = epad // (NW * EB)

    src = edge_index[0].astype(jnp.int32)
    dst = edge_index[1].astype(jnp.int32)
    pad_cfg = ((0, epad - n_edges),)
    srcr = jnp.pad(src, pad_cfg, constant_values=N_NODES).reshape(NW, kb, EB)
    dstr = jnp.pad(dst, pad_cfg, constant_values=N_NODES).reshape(NW, kb, EB)

    degp = _make_deg_kernel(kb)(dstr)

    dinv_c, y1 = pl.pallas_call(
        _tc_k1,
        grid=(NPAD // RB,),
        in_specs=[
            pl.BlockSpec((NC, RB, 1), lambda i: (0, i, 0)),
            pl.BlockSpec((RB, x.shape[1]), lambda i: (i, 0)),
            pl.BlockSpec(W1.shape, lambda i: (0, 0)),
        ],
        out_specs=(
            pl.BlockSpec((RB, 1), lambda i: (i, 0)),
            pl.BlockSpec((RB, H), lambda i: (i, 0)),
        ),
        out_shape=(
            jax.ShapeDtypeStruct((NPAD, 1), jnp.float32),
            jax.ShapeDtypeStruct((NPAD, H), jnp.float32),
        ),
    )(degp.reshape(NC, NPAD, 1), x, W1)
    dinv_flat = dinv_c.reshape(NPAD)

    p1, cpart = _make_agg_kernel(kb, do_c=True)(y1, srcr, dstr, dinv_flat)

    y2 = pl.pallas_call(
        _tc_k2,
        out_shape=jax.ShapeDtypeStruct((NPAD, H), jnp.float32),
    )(p1, y1, dinv_c, b1.reshape(1, H), W2)

    (p2,) = _make_agg_kernel(kb, do_c=False)(y2, srcr, dstr)

    out = pl.pallas_call(
        _tc_k3,
        out_shape=jax.ShapeDtypeStruct((1, lin_W.shape[1]), jnp.float32),
    )(p2, y2, dinv_c, b2.reshape(1, H), cpart.T, W3,
      b3.reshape(1, H), lin_W, lin_b.reshape(1, lin_W.shape[1]))
    return out


# speedup vs baseline: 58.1704x; 1.0011x over previous
import functools

import jax
import jax.numpy as jnp
from jax import lax
from jax.experimental import pallas as pl
from jax.experimental.pallas import tpu as pltpu
from jax.experimental.pallas import tpu_sc as plsc

N_NODES = 10000
H = 32
NC = 2
NS = 16
NW = NC * NS
EB = 128
NPAD = 10240
ROWS_PER_TILE = NPAD // NS


def _zero_f32_vec(ref, n):
    z = jnp.zeros((16,), jnp.float32)

    def body(i, _):
        ref[pl.ds(i * 16, 16)] = z
        return 0

    lax.fori_loop(0, n // 16, body, 0)


def _zero_f32_rows(ref, rows):
    z = jnp.zeros((16,), jnp.float32)

    def body(r, _):
        ref[r, pl.ds(0, 16)] = z
        ref[r, pl.ds(16, 16)] = z
        return 0

    lax.fori_loop(0, rows, body, 0)


def _make_deg_kernel(kb):
    mesh = plsc.VectorSubcoreMesh(core_axis_name="c", subcore_axis_name="s", num_cores=NC, num_subcores=NS)

    @functools.partial(
        pl.kernel,
        mesh=mesh,
        out_type=jax.ShapeDtypeStruct((NC, NPAD), jnp.float32),
        scratch_types=[
            pltpu.VMEM_SHARED((NPAD,), jnp.float32),
            pltpu.VMEM((kb, EB), jnp.int32),
            pltpu.VMEM((EB,), jnp.float32),
            pltpu.VMEM((EB,), jnp.float32),
        ],
    )
    def deg_kernel(dst_hbm, out_hbm, acc_sh, dst_v, ones_v, tmp_v):
        cid = lax.axis_index("c")
        sid = lax.axis_index("s")
        wid = cid * NS + sid
        one = jnp.ones((16,), jnp.float32)
        for g in range(EB // 16):
            ones_v[pl.ds(g * 16, 16)] = one
        _zero_f32_vec(tmp_v, EB)
        base = sid * ROWS_PER_TILE
        for r in range(ROWS_PER_TILE // EB):
            pltpu.sync_copy(tmp_v, acc_sh.at[pl.ds(base + r * EB, EB)])
        pltpu.sync_copy(dst_hbm.at[wid], dst_v)
        plsc.subcore_barrier()

        def body(j, _):
            pltpu.sync_copy(ones_v, acc_sh.at[dst_v.at[j]], add=True)
            return 0

        lax.fori_loop(0, kb, body, 0)
        plsc.subcore_barrier()
        for r in range(ROWS_PER_TILE // EB):
            off = base + r * EB
            pltpu.sync_copy(acc_sh.at[pl.ds(off, EB)], tmp_v)
            pltpu.sync_copy(tmp_v, out_hbm.at[cid].at[pl.ds(off, EB)])

    return deg_kernel


NBUF = 8


def _make_agg_kernel(kb, do_c):
    mesh = plsc.VectorSubcoreMesh(core_axis_name="c", subcore_axis_name="s", num_cores=NC, num_subcores=NS)
    out_type = [jax.ShapeDtypeStruct((NC, NPAD, H), jnp.float32)]
    scratch = [
        pltpu.VMEM_SHARED((NPAD, H), jnp.float32),
        pltpu.VMEM_SHARED((NPAD, H), jnp.float32),
        pltpu.VMEM((kb, EB), jnp.int32),
        pltpu.VMEM((kb, EB), jnp.int32),
        [pltpu.VMEM((EB, H), jnp.float32)] * NBUF,
        [pltpu.SemaphoreType.DMA] * NBUF,
        [pltpu.SemaphoreType.DMA] * NBUF,
    ]
    if do_c:
        out_type.append(jax.ShapeDtypeStruct((NW, NPAD), jnp.float32))
        scratch += [
            pltpu.VMEM((NPAD,), jnp.float32),
            pltpu.VMEM((NPAD,), jnp.float32),
        ]

    nstg = ROWS_PER_TILE // EB

    def body_fn(*refs):
        if do_c:
            (y_hbm, src_hbm, dst_hbm, dinv_hbm, p_hbm, c_hbm,
             acc_sh, y_sh, src_v, dst_v, rows, gsems, ssems,
             dinv_v, cacc_v) = refs
        else:
            (y_hbm, src_hbm, dst_hbm, p_hbm,
             acc_sh, y_sh, src_v, dst_v, rows, gsems, ssems) = refs
        cid = lax.axis_index("c")
        sid = lax.axis_index("s")
        wid = cid * NS + sid
        base = sid * ROWS_PER_TILE
        _zero_f32_rows(rows[NBUF - 1], EB)
        for r in range(nstg):
            off = base + r * EB
            pltpu.async_copy(y_hbm.at[pl.ds(off, EB)], rows[r], gsems[r])
            pltpu.async_copy(rows[NBUF - 1], acc_sh.at[pl.ds(off, EB)],
                             ssems[r])
        pltpu.sync_copy(src_hbm.at[wid], src_v)
        pltpu.sync_copy(dst_hbm.at[wid], dst_v)
        if do_c:
            pltpu.sync_copy(dinv_hbm, dinv_v)
            _zero_f32_vec(cacc_v, NPAD)
        for r in range(nstg):
            off = base + r * EB
            pltpu.make_async_copy(
                y_hbm.at[pl.ds(off, EB)], rows[r], gsems[r]).wait()
            pltpu.async_copy(rows[r], y_sh.at[pl.ds(off, EB)], gsems[r])
        for r in range(nstg):
            off = base + r * EB
            pltpu.make_async_copy(
                rows[r], y_sh.at[pl.ds(off, EB)], gsems[r]).wait()
            pltpu.make_async_copy(
                rows[NBUF - 1], acc_sh.at[pl.ds(off, EB)], ssems[r]).wait()
        plsc.subcore_barrier()

        for b in range(NBUF):
            pltpu.async_copy(y_sh.at[src_v.at[b]], rows[b], gsems[b])

        def body(g, _):
            j0 = g * NBUF
            for b in range(NBUF):
                j = j0 + b
                pltpu.make_async_copy(
                    y_sh.at[src_v.at[j]], rows[b], gsems[b]).wait()
                pltpu.async_copy(rows[b], acc_sh.at[dst_v.at[j]], ssems[b],
                                 add=True)
                if do_c:
                    for gg in range(EB // 16):
                        didx = dst_v[j, pl.ds(gg * 16, 16)]
                        sidx = src_v[j, pl.ds(gg * 16, 16)]
                        vals16 = plsc.load_gather(dinv_v, [didx])
                        plsc.addupdate_scatter(cacc_v, [sidx], vals16)
                nxt = j + NBUF

                @pl.when(nxt < kb)
                def _():
                    pltpu.make_async_copy(
                        rows[b], acc_sh.at[dst_v.at[j]], ssems[b]).wait()
                    pltpu.async_copy(y_sh.at[src_v.at[nxt]], rows[b],
                                     gsems[b])
            return 0

        lax.fori_loop(0, kb // NBUF, body, 0)
        if do_c:
            pltpu.async_copy(cacc_v, c_hbm.at[wid], gsems[NBUF - 1])
        last = kb - NBUF
        for b in range(NBUF):
            pltpu.make_async_copy(
                rows[b], acc_sh.at[dst_v.at[last + b]], ssems[b]).wait()
        plsc.subcore_barrier()
        for r in range(nstg):
            off = base + r * EB
            pltpu.sync_copy(acc_sh.at[pl.ds(off, EB)], rows[0])
            pltpu.sync_copy(rows[0], p_hbm.at[cid].at[pl.ds(off, EB)])
        if do_c:
            pltpu.make_async_copy(cacc_v, c_hbm.at[wid], gsems[NBUF - 1]).wait()

    return pl.kernel(
        body_fn, mesh=mesh, out_type=out_type, scratch_types=scratch,
        compiler_params=pltpu.CompilerParams(
            needs_layout_passes=False, use_tc_tiling_on_sc=False))


RB = 1280


def _tc_k0(x_ref, w1_ref, xw_ref):
    xw_ref[...] = jnp.dot(x_ref[...], w1_ref[...],
                          preferred_element_type=jnp.float32)


def _tc_k1(degp_ref, xw_ref, dinv_ref, y1_ref):
    i = pl.program_id(0)
    iota = i * RB + lax.broadcasted_iota(jnp.int32, (RB, 1), 0)
    valid = iota < N_NODES
    deg = degp_ref[0] + degp_ref[1] + 1.0
    dinv = jnp.where(valid, lax.rsqrt(deg), 0.0)
    dinv_ref[...] = dinv
    y1_ref[...] = dinv * xw_ref[...]


def _tc_k2(p_ref, y_ref, dinv_ref, b_ref, w_ref, out_ref):
    dinv = dinv_ref[...]
    h = jax.nn.relu(dinv * (p_ref[0] + p_ref[1] + y_ref[...]) + b_ref[...])
    out_ref[...] = dinv * jnp.dot(h, w_ref[...],
                                  preferred_element_type=jnp.float32)


def _tc_k3(p_ref, y_ref, dinv_ref, b2_ref, cp_ref, w3_ref, b3_ref,
           lw_ref, lb_ref, out_ref):
    dinv = dinv_ref[...]
    h2 = jax.nn.relu(dinv * (p_ref[0] + p_ref[1] + y_ref[...]) + b2_ref[...])
    c = dinv * (jnp.sum(cp_ref[...], axis=1, keepdims=True) + dinv)
    ws = jnp.sum(c * h2, axis=0, keepdims=True)
    pooled = jnp.dot(ws, w3_ref[...],
                     preferred_element_type=jnp.float32) * (1.0 / N_NODES)
    pooled = pooled + b3_ref[...]
    logits = jnp.dot(pooled, lw_ref[...],
                     preferred_element_type=jnp.float32) + lb_ref[...]
    out_ref[...] = jax.nn.softmax(logits, axis=1)


def kernel(x, edge_index, W1, b1, W2, b2, W3, b3, lin_W, lin_b):
    n_edges = edge_index.shape[1]
    chunk = NW * EB * NBUF
    epad = -(-n_edges // chunk) * chunk
    kb = epad // (NW * EB)

    src = edge_index[0].astype(jnp.int32)
    dst = edge_index[1].astype(jnp.int32)
    pad_cfg = ((0, epad - n_edges),)
    srcr = jnp.pad(src, pad_cfg, constant_values=N_NODES).reshape(NW, kb, EB)
    dstr = jnp.pad(dst, pad_cfg, constant_values=N_NODES).reshape(NW, kb, EB)

    xw1 = pl.pallas_call(
        _tc_k0,
        grid=(NPAD // RB,),
        in_specs=[
            pl.BlockSpec((RB, x.shape[1]), lambda i: (i, 0)),
            pl.BlockSpec(W1.shape, lambda i: (0, 0)),
        ],
        out_specs=pl.BlockSpec((RB, H), lambda i: (i, 0)),
        out_shape=jax.ShapeDtypeStruct((NPAD, H), jnp.float32),
    )(x, W1)
    degp = _make_deg_kernel(kb)(dstr)

    dinv_c, y1 = pl.pallas_call(
        _tc_k1,
        grid=(NPAD // RB,),
        in_specs=[
            pl.BlockSpec((NC, RB, 1), lambda i: (0, i, 0)),
            pl.BlockSpec((RB, H), lambda i: (i, 0)),
        ],
        out_specs=(
            pl.BlockSpec((RB, 1), lambda i: (i, 0)),
            pl.BlockSpec((RB, H), lambda i: (i, 0)),
        ),
        out_shape=(
            jax.ShapeDtypeStruct((NPAD, 1), jnp.float32),
            jax.ShapeDtypeStruct((NPAD, H), jnp.float32),
        ),
    )(degp.reshape(NC, NPAD, 1), xw1)
    dinv_flat = dinv_c.reshape(NPAD)

    p1, cpart = _make_agg_kernel(kb, do_c=True)(y1, srcr, dstr, dinv_flat)

    y2 = pl.pallas_call(
        _tc_k2,
        out_shape=jax.ShapeDtypeStruct((NPAD, H), jnp.float32),
    )(p1, y1, dinv_c, b1.reshape(1, H), W2)

    (p2,) = _make_agg_kernel(kb, do_c=False)(y2, srcr, dstr)

    out = pl.pallas_call(
        _tc_k3,
        out_shape=jax.ShapeDtypeStruct((1, lin_W.shape[1]), jnp.float32),
    )(p2, y2, dinv_c, b2.reshape(1, H), cpart.T, W3,
      b3.reshape(1, H), lin_W, lin_b.reshape(1, lin_W.shape[1]))
    return out


# speedup vs baseline: 59.6306x; 1.0251x over previous
import functools

import jax
import jax.numpy as jnp
from jax import lax
from jax.experimental import pallas as pl
from jax.experimental.pallas import tpu as pltpu
from jax.experimental.pallas import tpu_sc as plsc

N_NODES = 10000
H = 32
NC = 2
NS = 16
NW = NC * NS
EB = 128
NPAD = 10240
ROWS_PER_TILE = NPAD // NS


def _zero_f32_vec(ref, n):
    z = jnp.zeros((16,), jnp.float32)

    def body(i, _):
        for u in range(8):
            ref[pl.ds(i * 128 + u * 16, 16)] = z
        return 0

    lax.fori_loop(0, n // 128, body, 0)


def _zero_f32_rows(ref, rows):
    z = jnp.zeros((16,), jnp.float32)

    def body(r, _):
        for u in range(4):
            ref[r * 4 + u, pl.ds(0, 16)] = z
            ref[r * 4 + u, pl.ds(16, 16)] = z
        return 0

    lax.fori_loop(0, rows // 4, body, 0)


def _make_deg_kernel(kb):
    mesh = plsc.VectorSubcoreMesh(core_axis_name="c", subcore_axis_name="s", num_cores=NC, num_subcores=NS)

    @functools.partial(
        pl.kernel,
        mesh=mesh,
        out_type=jax.ShapeDtypeStruct((NC, NPAD), jnp.float32),
        scratch_types=[
            pltpu.VMEM_SHARED((NPAD,), jnp.float32),
            pltpu.VMEM((kb, EB), jnp.int32),
            pltpu.VMEM((EB,), jnp.float32),
            pltpu.VMEM((EB,), jnp.float32),
        ],
    )
    def deg_kernel(dst_hbm, out_hbm, acc_sh, dst_v, ones_v, tmp_v):
        cid = lax.axis_index("c")
        sid = lax.axis_index("s")
        wid = cid * NS + sid
        one = jnp.ones((16,), jnp.float32)
        for g in range(EB // 16):
            ones_v[pl.ds(g * 16, 16)] = one
        _zero_f32_vec(tmp_v, EB)
        base = sid * ROWS_PER_TILE
        for r in range(ROWS_PER_TILE // EB):
            pltpu.sync_copy(tmp_v, acc_sh.at[pl.ds(base + r * EB, EB)])
        pltpu.sync_copy(dst_hbm.at[wid], dst_v)
        plsc.subcore_barrier()

        def body(j, _):
            pltpu.sync_copy(ones_v, acc_sh.at[dst_v.at[j]], add=True)
            return 0

        lax.fori_loop(0, kb, body, 0)
        plsc.subcore_barrier()
        for r in range(ROWS_PER_TILE // EB):
            off = base + r * EB
            pltpu.sync_copy(acc_sh.at[pl.ds(off, EB)], tmp_v)
            pltpu.sync_copy(tmp_v, out_hbm.at[cid].at[pl.ds(off, EB)])

    return deg_kernel


NBUF = 8


def _make_agg_kernel(kb, do_c):
    mesh = plsc.VectorSubcoreMesh(core_axis_name="c", subcore_axis_name="s", num_cores=NC, num_subcores=NS)
    out_type = [jax.ShapeDtypeStruct((NC, NPAD, H), jnp.float32)]
    scratch = [
        pltpu.VMEM_SHARED((NPAD, H), jnp.float32),
        pltpu.VMEM_SHARED((NPAD, H), jnp.float32),
        pltpu.VMEM((kb, EB), jnp.int32),
        pltpu.VMEM((kb, EB), jnp.int32),
        [pltpu.VMEM((EB, H), jnp.float32)] * NBUF,
        [pltpu.SemaphoreType.DMA] * NBUF,
        [pltpu.SemaphoreType.DMA] * NBUF,
    ]
    if do_c:
        out_type.append(jax.ShapeDtypeStruct((NW, NPAD), jnp.float32))
        scratch += [
            pltpu.VMEM((NPAD,), jnp.float32),
            pltpu.VMEM((NPAD,), jnp.float32),
        ]

    nstg = ROWS_PER_TILE // EB

    def body_fn(*refs):
        if do_c:
            (y_hbm, src_hbm, dst_hbm, dinv_hbm, p_hbm, c_hbm,
             acc_sh, y_sh, src_v, dst_v, rows, gsems, ssems,
             dinv_v, cacc_v) = refs
        else:
            (y_hbm, src_hbm, dst_hbm, p_hbm,
             acc_sh, y_sh, src_v, dst_v, rows, gsems, ssems) = refs
        cid = lax.axis_index("c")
        sid = lax.axis_index("s")
        wid = cid * NS + sid
        base = sid * ROWS_PER_TILE
        _zero_f32_rows(rows[NBUF - 1], EB)
        for r in range(nstg):
            off = base + r * EB
            pltpu.async_copy(y_hbm.at[pl.ds(off, EB)], rows[r], gsems[r])
            pltpu.async_copy(rows[NBUF - 1], acc_sh.at[pl.ds(off, EB)],
                             ssems[r])
        pltpu.sync_copy(src_hbm.at[wid], src_v)
        pltpu.sync_copy(dst_hbm.at[wid], dst_v)
        if do_c:
            pltpu.sync_copy(dinv_hbm, dinv_v)
            _zero_f32_vec(cacc_v, NPAD)
        for r in range(nstg):
            off = base + r * EB
            pltpu.make_async_copy(
                y_hbm.at[pl.ds(off, EB)], rows[r], gsems[r]).wait()
            pltpu.async_copy(rows[r], y_sh.at[pl.ds(off, EB)], gsems[r])
        for r in range(nstg):
            off = base + r * EB
            pltpu.make_async_copy(
                rows[r], y_sh.at[pl.ds(off, EB)], gsems[r]).wait()
            pltpu.make_async_copy(
                rows[NBUF - 1], acc_sh.at[pl.ds(off, EB)], ssems[r]).wait()
        plsc.subcore_barrier()

        for b in range(NBUF):
            pltpu.async_copy(y_sh.at[src_v.at[b]], rows[b], gsems[b])

        def body(g, _):
            j0 = g * NBUF
            for b in range(NBUF):
                j = j0 + b
                pltpu.make_async_copy(
                    y_sh.at[src_v.at[j]], rows[b], gsems[b]).wait()
                pltpu.async_copy(rows[b], acc_sh.at[dst_v.at[j]], ssems[b],
                                 add=True)
                if do_c:
                    for gg in range(EB // 16):
                        didx = dst_v[j, pl.ds(gg * 16, 16)]
                        sidx = src_v[j, pl.ds(gg * 16, 16)]
                        vals16 = plsc.load_gather(dinv_v, [didx])
                        plsc.addupdate_scatter(cacc_v, [sidx], vals16)
                nxt = j + NBUF

                @pl.when(nxt < kb)
                def _():
                    pltpu.make_async_copy(
                        rows[b], acc_sh.at[dst_v.at[j]], ssems[b]).wait()
                    pltpu.async_copy(y_sh.at[src_v.at[nxt]], rows[b],
                                     gsems[b])
            return 0

        lax.fori_loop(0, kb // NBUF, body, 0)
        if do_c:
            pltpu.async_copy(cacc_v, c_hbm.at[wid], gsems[NBUF - 1])
        last = kb - NBUF
        for b in range(NBUF):
            pltpu.make_async_copy(
                rows[b], acc_sh.at[dst_v.at[last + b]], ssems[b]).wait()
        plsc.subcore_barrier()
        for r in range(nstg):
            off = base + r * EB
            pltpu.async_copy(acc_sh.at[pl.ds(off, EB)], rows[r], gsems[r])
        for r in range(nstg):
            off = base + r * EB
            pltpu.make_async_copy(
                acc_sh.at[pl.ds(off, EB)], rows[r], gsems[r]).wait()
            pltpu.async_copy(rows[r], p_hbm.at[cid].at[pl.ds(off, EB)],
                             ssems[r])
        for r in range(nstg):
            off = base + r * EB
            pltpu.make_async_copy(
                rows[r], p_hbm.at[cid].at[pl.ds(off, EB)], ssems[r]).wait()
        if do_c:
            pltpu.make_async_copy(cacc_v, c_hbm.at[wid], gsems[NBUF - 1]).wait()

    return pl.kernel(
        body_fn, mesh=mesh, out_type=out_type, scratch_types=scratch,
        compiler_params=pltpu.CompilerParams(
            needs_layout_passes=False, use_tc_tiling_on_sc=False))


RB = 1280


def _tc_k0(x_ref, w1_ref, xw_ref):
    xw_ref[...] = jnp.dot(x_ref[...], w1_ref[...],
                          preferred_element_type=jnp.float32)


def _tc_k1(degp_ref, xw_ref, dinv_ref, y1_ref):
    i = pl.program_id(0)
    iota = i * RB + lax.broadcasted_iota(jnp.int32, (RB, 1), 0)
    valid = iota < N_NODES
    deg = degp_ref[0] + degp_ref[1] + 1.0
    dinv = jnp.where(valid, lax.rsqrt(deg), 0.0)
    dinv_ref[...] = dinv
    y1_ref[...] = dinv * xw_ref[...]


def _tc_k2(p_ref, y_ref, dinv_ref, b_ref, w_ref, out_ref):
    dinv = dinv_ref[...]
    h = jax.nn.relu(dinv * (p_ref[0] + p_ref[1] + y_ref[...]) + b_ref[...])
    out_ref[...] = dinv * jnp.dot(h, w_ref[...],
                                  preferred_element_type=jnp.float32)


def _tc_k3(p_ref, y_ref, dinv_ref, b2_ref, cp_ref, w3_ref, b3_ref,
           lw_ref, lb_ref, out_ref):
    dinv = dinv_ref[...]
    h2 = jax.nn.relu(dinv * (p_ref[0] + p_ref[1] + y_ref[...]) + b2_ref[...])
    c = dinv * (jnp.sum(cp_ref[...], axis=1, keepdims=True) + dinv)
    ws = jnp.sum(c * h2, axis=0, keepdims=True)
    pooled = jnp.dot(ws, w3_ref[...],
                     preferred_element_type=jnp.float32) * (1.0 / N_NODES)
    pooled = pooled + b3_ref[...]
    logits = jnp.dot(pooled, lw_ref[...],
                     preferred_element_type=jnp.float32) + lb_ref[...]
    out_ref[...] = jax.nn.softmax(logits, axis=1)


def kernel(x, edge_index, W1, b1, W2, b2, W3, b3, lin_W, lin_b):
    n_edges = edge_index.shape[1]
    chunk = NW * EB * NBUF
    epad = -(-n_edges // chunk) * chunk
    kb = epad // (NW * EB)

    src = edge_index[0].astype(jnp.int32)
    dst = edge_index[1].astype(jnp.int32)
    pad_cfg = ((0, epad - n_edges),)
    srcr = jnp.pad(src, pad_cfg, constant_values=N_NODES).reshape(NW, kb, EB)
    dstr = jnp.pad(dst, pad_cfg, constant_values=N_NODES).reshape(NW, kb, EB)

    xw1 = pl.pallas_call(
        _tc_k0,
        grid=(NPAD // RB,),
        in_specs=[
            pl.BlockSpec((RB, x.shape[1]), lambda i: (i, 0)),
            pl.BlockSpec(W1.shape, lambda i: (0, 0)),
        ],
        out_specs=pl.BlockSpec((RB, H), lambda i: (i, 0)),
        out_shape=jax.ShapeDtypeStruct((NPAD, H), jnp.float32),
    )(x, W1)
    degp = _make_deg_kernel(kb)(dstr)

    dinv_c, y1 = pl.pallas_call(
        _tc_k1,
        grid=(NPAD // RB,),
        in_specs=[
            pl.BlockSpec((NC, RB, 1), lambda i: (0, i, 0)),
            pl.BlockSpec((RB, H), lambda i: (i, 0)),
        ],
        out_specs=(
            pl.BlockSpec((RB, 1), lambda i: (i, 0)),
            pl.BlockSpec((RB, H), lambda i: (i, 0)),
        ),
        out_shape=(
            jax.ShapeDtypeStruct((NPAD, 1), jnp.float32),
            jax.ShapeDtypeStruct((NPAD, H), jnp.float32),
        ),
    )(degp.reshape(NC, NPAD, 1), xw1)
    dinv_flat = dinv_c.reshape(NPAD)

    p1, cpart = _make_agg_kernel(kb, do_c=True)(y1, srcr, dstr, dinv_flat)

    y2 = pl.pallas_call(
        _tc_k2,
        out_shape=jax.ShapeDtypeStruct((NPAD, H), jnp.float32),
    )(p1, y1, dinv_c, b1.reshape(1, H), W2)

    (p2,) = _make_agg_kernel(kb, do_c=False)(y2, srcr, dstr)

    out = pl.pallas_call(
        _tc_k3,
        out_shape=jax.ShapeDtypeStruct((1, lin_W.shape[1]), jnp.float32),
    )(p2, y2, dinv_c, b2.reshape(1, H), cpart.T, W3,
      b3.reshape(1, H), lin_W, lin_b.reshape(1, lin_W.shape[1]))
    return out


# speedup vs baseline: 62.1318x; 1.0419x over previous
import functools

import jax
import jax.numpy as jnp
from jax import lax
from jax.experimental import pallas as pl
from jax.experimental.pallas import tpu as pltpu
from jax.experimental.pallas import tpu_sc as plsc

N_NODES = 10000
H = 32
NC = 2
NS = 16
NW = NC * NS
EB = 128
NPAD = 10240
ROWS_PER_TILE = NPAD // NS


def _zero_f32_vec(ref, n):
    z = jnp.zeros((16,), jnp.float32)

    def body(i, _):
        for u in range(8):
            ref[pl.ds(i * 128 + u * 16, 16)] = z
        return 0

    lax.fori_loop(0, n // 128, body, 0)


def _zero_f32_rows(ref, rows):
    z = jnp.zeros((16,), jnp.float32)

    def body(r, _):
        for u in range(4):
            ref[r * 4 + u, pl.ds(0, 16)] = z
            ref[r * 4 + u, pl.ds(16, 16)] = z
        return 0

    lax.fori_loop(0, rows // 4, body, 0)


def _make_deg_kernel(kb):
    mesh = plsc.VectorSubcoreMesh(core_axis_name="c", subcore_axis_name="s", num_cores=NC, num_subcores=NS)

    @functools.partial(
        pl.kernel,
        mesh=mesh,
        out_type=jax.ShapeDtypeStruct((NC, NPAD), jnp.float32),
        scratch_types=[
            pltpu.VMEM_SHARED((NPAD,), jnp.float32),
            pltpu.VMEM((kb, EB), jnp.int32),
            pltpu.VMEM((EB,), jnp.float32),
            pltpu.VMEM((EB,), jnp.float32),
        ],
    )
    def deg_kernel(dst_hbm, out_hbm, acc_sh, dst_v, ones_v, tmp_v):
        cid = lax.axis_index("c")
        sid = lax.axis_index("s")
        wid = cid * NS + sid
        one = jnp.ones((16,), jnp.float32)
        for g in range(EB // 16):
            ones_v[pl.ds(g * 16, 16)] = one
        _zero_f32_vec(tmp_v, EB)
        base = sid * ROWS_PER_TILE
        for r in range(ROWS_PER_TILE // EB):
            pltpu.sync_copy(tmp_v, acc_sh.at[pl.ds(base + r * EB, EB)])
        pltpu.sync_copy(dst_hbm.at[wid], dst_v)
        plsc.subcore_barrier()

        def body(j, _):
            pltpu.sync_copy(ones_v, acc_sh.at[dst_v.at[j]], add=True)
            return 0

        lax.fori_loop(0, kb, body, 0)
        plsc.subcore_barrier()
        for r in range(ROWS_PER_TILE // EB):
            off = base + r * EB
            pltpu.sync_copy(acc_sh.at[pl.ds(off, EB)], tmp_v)
            pltpu.sync_copy(tmp_v, out_hbm.at[cid].at[pl.ds(off, EB)])

    return deg_kernel


NBUF = 8


def _make_agg_kernel(kb, do_c):
    mesh = plsc.VectorSubcoreMesh(core_axis_name="c", subcore_axis_name="s", num_cores=NC, num_subcores=NS)
    out_type = [jax.ShapeDtypeStruct((NC, NPAD, H), jnp.float32)]
    scratch = [
        pltpu.VMEM_SHARED((NPAD, H), jnp.float32),
        pltpu.VMEM_SHARED((NPAD, H), jnp.float32),
        pltpu.VMEM((kb, EB), jnp.int32),
        pltpu.VMEM((kb, EB), jnp.int32),
        [pltpu.VMEM((EB, H), jnp.float32)] * NBUF,
        [pltpu.SemaphoreType.DMA] * NBUF,
        [pltpu.SemaphoreType.DMA] * NBUF,
    ]
    if do_c:
        out_type.append(jax.ShapeDtypeStruct((NW, NPAD), jnp.float32))
        scratch += [
            pltpu.VMEM((NPAD,), jnp.float32),
            pltpu.VMEM((NPAD,), jnp.float32),
        ]

    nstg = ROWS_PER_TILE // EB

    def body_fn(*refs):
        if do_c:
            (y_hbm, src_hbm, dst_hbm, dinv_hbm, p_hbm, c_hbm,
             acc_sh, y_sh, src_v, dst_v, rows, gsems, ssems,
             dinv_v, cacc_v) = refs
        else:
            (y_hbm, src_hbm, dst_hbm, p_hbm,
             acc_sh, y_sh, src_v, dst_v, rows, gsems, ssems) = refs
        cid = lax.axis_index("c")
        sid = lax.axis_index("s")
        wid = cid * NS + sid
        base = sid * ROWS_PER_TILE
        _zero_f32_rows(rows[NBUF - 1], EB)
        for r in range(nstg):
            off = base + r * EB
            pltpu.async_copy(y_hbm.at[pl.ds(off, EB)], rows[r], gsems[r])
            pltpu.async_copy(rows[NBUF - 1], acc_sh.at[pl.ds(off, EB)],
                             ssems[r])
        pltpu.sync_copy(src_hbm.at[wid], src_v)
        pltpu.sync_copy(dst_hbm.at[wid], dst_v)
        if do_c:
            pltpu.sync_copy(dinv_hbm, dinv_v)
            _zero_f32_vec(cacc_v, NPAD)
        for r in range(nstg):
            off = base + r * EB
            pltpu.make_async_copy(
                y_hbm.at[pl.ds(off, EB)], rows[r], gsems[r]).wait()
            pltpu.async_copy(rows[r], y_sh.at[pl.ds(off, EB)], gsems[r])
        for r in range(nstg):
            off = base + r * EB
            pltpu.make_async_copy(
                rows[r], y_sh.at[pl.ds(off, EB)], gsems[r]).wait()
            pltpu.make_async_copy(
                rows[NBUF - 1], acc_sh.at[pl.ds(off, EB)], ssems[r]).wait()
        plsc.subcore_barrier()

        for b in range(NBUF):
            pltpu.async_copy(y_sh.at[src_v.at[b]], rows[b], gsems[b])

        def body(g, _):
            j0 = g * NBUF
            for b in range(NBUF):
                j = j0 + b
                pltpu.make_async_copy(
                    y_sh.at[src_v.at[j]], rows[b], gsems[b]).wait()
                pltpu.async_copy(rows[b], acc_sh.at[dst_v.at[j]], ssems[b],
                                 add=True)
                if do_c:
                    for gg in range(EB // 16):
                        didx = dst_v[j, pl.ds(gg * 16, 16)]
                        sidx = src_v[j, pl.ds(gg * 16, 16)]
                        vals16 = plsc.load_gather(dinv_v, [didx])
                        plsc.addupdate_scatter(cacc_v, [sidx], vals16)
                nxt = j + NBUF

                @pl.when(nxt < kb)
                def _():
                    pltpu.make_async_copy(
                        rows[b], acc_sh.at[dst_v.at[j]], ssems[b]).wait()
                    pltpu.async_copy(y_sh.at[src_v.at[nxt]], rows[b],
                                     gsems[b])
            return 0

        lax.fori_loop(0, kb // NBUF, body, 0)
        if do_c:
            pltpu.async_copy(cacc_v, c_hbm.at[wid], gsems[NBUF - 1])
        last = kb - NBUF
        for b in range(NBUF):
            pltpu.make_async_copy(
                rows[b], acc_sh.at[dst_v.at[last + b]], ssems[b]).wait()
        plsc.subcore_barrier()
        for r in range(nstg):
            off = base + r * EB
            pltpu.async_copy(acc_sh.at[pl.ds(off, EB)], rows[r], gsems[r])
        for r in range(nstg):
            off = base + r * EB
            pltpu.make_async_copy(
                acc_sh.at[pl.ds(off, EB)], rows[r], gsems[r]).wait()
            pltpu.async_copy(rows[r], p_hbm.at[cid].at[pl.ds(off, EB)],
                             ssems[r])
        for r in range(nstg):
            off = base + r * EB
            pltpu.make_async_copy(
                rows[r], p_hbm.at[cid].at[pl.ds(off, EB)], ssems[r]).wait()
        if do_c:
            pltpu.make_async_copy(cacc_v, c_hbm.at[wid], gsems[NBUF - 1]).wait()

    return pl.kernel(
        body_fn, mesh=mesh, out_type=out_type, scratch_types=scratch,
        compiler_params=pltpu.CompilerParams(
            needs_layout_passes=False, use_tc_tiling_on_sc=False))


RB = 1280


def _tc_k0(x_ref, w1_ref, xw_ref):
    xw_ref[...] = jnp.dot(x_ref[...], w1_ref[...],
                          preferred_element_type=jnp.float32)


def _tc_k1(degp_ref, xw_ref, dinv_ref, y1_ref):
    i = pl.program_id(0)
    iota = i * RB + lax.broadcasted_iota(jnp.int32, (1, RB), 1)
    valid = iota < N_NODES
    deg = degp_ref[0:1, :] + degp_ref[1:2, :] + 1.0
    dinv_row = jnp.where(valid, lax.rsqrt(deg), 0.0)
    dinv = dinv_row.reshape(RB, 1)
    dinv_ref[...] = dinv
    y1_ref[...] = dinv * xw_ref[...]


def _tc_k2(p_ref, y_ref, dinv_ref, b_ref, w_ref, out_ref):
    dinv = dinv_ref[...]
    h = jax.nn.relu(dinv * (p_ref[0] + p_ref[1] + y_ref[...]) + b_ref[...])
    out_ref[...] = dinv * jnp.dot(h, w_ref[...],
                                  preferred_element_type=jnp.float32)


def _tc_k3(p_ref, y_ref, dinv_ref, b2_ref, cp_ref, w3_ref, b3_ref,
           lw_ref, lb_ref, out_ref):
    dinv = dinv_ref[...]
    h2 = jax.nn.relu(dinv * (p_ref[0] + p_ref[1] + y_ref[...]) + b2_ref[...])
    c = dinv * (jnp.sum(cp_ref[...], axis=1, keepdims=True) + dinv)
    ws = jnp.sum(c * h2, axis=0, keepdims=True)
    pooled = jnp.dot(ws, w3_ref[...],
                     preferred_element_type=jnp.float32) * (1.0 / N_NODES)
    pooled = pooled + b3_ref[...]
    logits = jnp.dot(pooled, lw_ref[...],
                     preferred_element_type=jnp.float32) + lb_ref[...]
    out_ref[...] = jax.nn.softmax(logits, axis=1)


def kernel(x, edge_index, W1, b1, W2, b2, W3, b3, lin_W, lin_b):
    n_edges = edge_index.shape[1]
    chunk = NW * EB * NBUF
    epad = -(-n_edges // chunk) * chunk
    kb = epad // (NW * EB)

    src = edge_index[0].astype(jnp.int32)
    dst = edge_index[1].astype(jnp.int32)
    pad_cfg = ((0, epad - n_edges),)
    srcr = jnp.pad(src, pad_cfg, constant_values=N_NODES).reshape(NW, kb, EB)
    dstr = jnp.pad(dst, pad_cfg, constant_values=N_NODES).reshape(NW, kb, EB)

    xw1 = pl.pallas_call(
        _tc_k0,
        grid=(NPAD // RB,),
        in_specs=[
            pl.BlockSpec((RB, x.shape[1]), lambda i: (i, 0)),
            pl.BlockSpec(W1.shape, lambda i: (0, 0)),
        ],
        out_specs=pl.BlockSpec((RB, H), lambda i: (i, 0)),
        out_shape=jax.ShapeDtypeStruct((NPAD, H), jnp.float32),
    )(x, W1)
    degp = _make_deg_kernel(kb)(dstr)

    dinv_c, y1 = pl.pallas_call(
        _tc_k1,
        grid=(NPAD // RB,),
        in_specs=[
            pl.BlockSpec((NC, RB), lambda i: (0, i)),
            pl.BlockSpec((RB, H), lambda i: (i, 0)),
        ],
        out_specs=(
            pl.BlockSpec((RB, 1), lambda i: (i, 0)),
            pl.BlockSpec((RB, H), lambda i: (i, 0)),
        ),
        out_shape=(
            jax.ShapeDtypeStruct((NPAD, 1), jnp.float32),
            jax.ShapeDtypeStruct((NPAD, H), jnp.float32),
        ),
    )(degp, xw1)
    dinv_flat = dinv_c.reshape(NPAD)

    p1, cpart = _make_agg_kernel(kb, do_c=True)(y1, srcr, dstr, dinv_flat)

    y2 = pl.pallas_call(
        _tc_k2,
        out_shape=jax.ShapeDtypeStruct((NPAD, H), jnp.float32),
    )(p1, y1, dinv_c, b1.reshape(1, H), W2)

    (p2,) = _make_agg_kernel(kb, do_c=False)(y2, srcr, dstr)

    out = pl.pallas_call(
        _tc_k3,
        out_shape=jax.ShapeDtypeStruct((1, lin_W.shape[1]), jnp.float32),
    )(p2, y2, dinv_c, b2.reshape(1, H), cpart.T, W3,
      b3.reshape(1, H), lin_W, lin_b.reshape(1, lin_W.shape[1]))
    return out


# speedup vs baseline: 63.5485x; 1.0228x over previous
import functools

import jax
import jax.numpy as jnp
from jax import lax
from jax.experimental import pallas as pl
from jax.experimental.pallas import tpu as pltpu
from jax.experimental.pallas import tpu_sc as plsc

N_NODES = 10000
H = 32
NC = 2
NS = 16
NW = NC * NS
EB = 128
NPAD = 10240
ROWS_PER_TILE = NPAD // NS


def _zero_f32_vec(ref, n):
    z = jnp.zeros((16,), jnp.float32)

    def body(i, _):
        for u in range(8):
            ref[pl.ds(i * 128 + u * 16, 16)] = z
        return 0

    lax.fori_loop(0, n // 128, body, 0)


def _zero_f32_rows(ref, rows):
    z = jnp.zeros((16,), jnp.float32)

    def body(r, _):
        for u in range(4):
            ref[r * 4 + u, pl.ds(0, 16)] = z
            ref[r * 4 + u, pl.ds(16, 16)] = z
        return 0

    lax.fori_loop(0, rows // 4, body, 0)


def _make_deg_kernel(kb):
    mesh = plsc.VectorSubcoreMesh(core_axis_name="c", subcore_axis_name="s", num_cores=NC, num_subcores=NS)

    @functools.partial(
        pl.kernel,
        mesh=mesh,
        out_type=jax.ShapeDtypeStruct((NC, NPAD), jnp.float32),
        scratch_types=[
            pltpu.VMEM_SHARED((NPAD,), jnp.float32),
            pltpu.VMEM((kb, EB), jnp.int32),
            pltpu.VMEM((EB,), jnp.float32),
            pltpu.VMEM((EB,), jnp.float32),
        ],
    )
    def deg_kernel(dst_hbm, out_hbm, acc_sh, dst_v, ones_v, tmp_v):
        cid = lax.axis_index("c")
        sid = lax.axis_index("s")
        wid = cid * NS + sid
        one = jnp.ones((16,), jnp.float32)
        for g in range(EB // 16):
            ones_v[pl.ds(g * 16, 16)] = one
        _zero_f32_vec(tmp_v, EB)
        base = sid * ROWS_PER_TILE
        for r in range(ROWS_PER_TILE // EB):
            pltpu.sync_copy(tmp_v, acc_sh.at[pl.ds(base + r * EB, EB)])
        pltpu.sync_copy(dst_hbm.at[wid], dst_v)
        plsc.subcore_barrier()

        def body(j, _):
            pltpu.sync_copy(ones_v, acc_sh.at[dst_v.at[j]], add=True)
            return 0

        lax.fori_loop(0, kb, body, 0)
        plsc.subcore_barrier()
        for r in range(ROWS_PER_TILE // EB):
            off = base + r * EB
            pltpu.sync_copy(acc_sh.at[pl.ds(off, EB)], tmp_v)
            pltpu.sync_copy(tmp_v, out_hbm.at[cid].at[pl.ds(off, EB)])

    return deg_kernel


NBUF = 8


def _make_agg_kernel(kb, do_c):
    mesh = plsc.VectorSubcoreMesh(core_axis_name="c", subcore_axis_name="s", num_cores=NC, num_subcores=NS)
    out_type = [jax.ShapeDtypeStruct((NC, NPAD, H), jnp.float32)]
    scratch = [
        pltpu.VMEM_SHARED((NPAD, H), jnp.float32),
        pltpu.VMEM_SHARED((NPAD, H), jnp.float32),
        pltpu.VMEM((kb, EB), jnp.int32),
        pltpu.VMEM((kb, EB), jnp.int32),
        [pltpu.VMEM((EB, H), jnp.float32)] * NBUF,
        [pltpu.SemaphoreType.DMA] * NBUF,
        [pltpu.SemaphoreType.DMA] * NBUF,
    ]
    if do_c:
        out_type.append(jax.ShapeDtypeStruct((NW, NPAD), jnp.float32))
        scratch += [
            pltpu.VMEM((NPAD,), jnp.float32),
            pltpu.VMEM((NPAD,), jnp.float32),
        ]

    nstg = ROWS_PER_TILE // EB

    def body_fn(*refs):
        if do_c:
            (y_hbm, src_hbm, dst_hbm, dinv_hbm, p_hbm, c_hbm,
             acc_sh, y_sh, src_v, dst_v, rows, gsems, ssems,
             dinv_v, cacc_v) = refs
        else:
            (y_hbm, src_hbm, dst_hbm, p_hbm,
             acc_sh, y_sh, src_v, dst_v, rows, gsems, ssems) = refs
        cid = lax.axis_index("c")
        sid = lax.axis_index("s")
        wid = cid * NS + sid
        base = sid * ROWS_PER_TILE
        _zero_f32_rows(rows[NBUF - 1], EB)
        for r in range(nstg):
            off = base + r * EB
            pltpu.async_copy(y_hbm.at[pl.ds(off, EB)], rows[r], gsems[r])
            pltpu.async_copy(rows[NBUF - 1], acc_sh.at[pl.ds(off, EB)],
                             ssems[r])
        pltpu.sync_copy(src_hbm.at[wid], src_v)
        pltpu.sync_copy(dst_hbm.at[wid], dst_v)
        if do_c:
            pltpu.sync_copy(dinv_hbm, dinv_v)
            _zero_f32_vec(cacc_v, NPAD)
        for r in range(nstg):
            off = base + r * EB
            pltpu.make_async_copy(
                y_hbm.at[pl.ds(off, EB)], rows[r], gsems[r]).wait()
            pltpu.async_copy(rows[r], y_sh.at[pl.ds(off, EB)], gsems[r])
        for r in range(nstg):
            off = base + r * EB
            pltpu.make_async_copy(
                rows[r], y_sh.at[pl.ds(off, EB)], gsems[r]).wait()
            pltpu.make_async_copy(
                rows[NBUF - 1], acc_sh.at[pl.ds(off, EB)], ssems[r]).wait()
        plsc.subcore_barrier()

        for b in range(NBUF):
            pltpu.async_copy(y_sh.at[src_v.at[b]], rows[b], gsems[b])

        def body(g, _):
            j0 = g * NBUF
            for b in range(NBUF):
                j = j0 + b
                pltpu.make_async_copy(
                    y_sh.at[src_v.at[j]], rows[b], gsems[b]).wait()
                pltpu.async_copy(rows[b], acc_sh.at[dst_v.at[j]], ssems[b],
                                 add=True)
                if do_c:
                    for gg in range(EB // 16):
                        didx = dst_v[j, pl.ds(gg * 16, 16)]
                        sidx = src_v[j, pl.ds(gg * 16, 16)]
                        vals16 = plsc.load_gather(dinv_v, [didx])
                        plsc.addupdate_scatter(cacc_v, [sidx], vals16)
                nxt = j + NBUF

                @pl.when(nxt < kb)
                def _():
                    pltpu.make_async_copy(
                        rows[b], acc_sh.at[dst_v.at[j]], ssems[b]).wait()
                    pltpu.async_copy(y_sh.at[src_v.at[nxt]], rows[b],
                                     gsems[b])
            return 0

        lax.fori_loop(0, kb // NBUF, body, 0)
        if do_c:
            pltpu.async_copy(cacc_v, c_hbm.at[wid], gsems[NBUF - 1])
        last = kb - NBUF
        for b in range(NBUF):
            pltpu.make_async_copy(
                rows[b], acc_sh.at[dst_v.at[last + b]], ssems[b]).wait()
        plsc.subcore_barrier()
        for r in range(nstg):
            off = base + r * EB
            pltpu.async_copy(acc_sh.at[pl.ds(off, EB)], rows[r], gsems[r])
        for r in range(nstg):
            off = base + r * EB
            pltpu.make_async_copy(
                acc_sh.at[pl.ds(off, EB)], rows[r], gsems[r]).wait()
            pltpu.async_copy(rows[r], p_hbm.at[cid].at[pl.ds(off, EB)],
                             ssems[r])
        for r in range(nstg):
            off = base + r * EB
            pltpu.make_async_copy(
                rows[r], p_hbm.at[cid].at[pl.ds(off, EB)], ssems[r]).wait()
        if do_c:
            pltpu.make_async_copy(cacc_v, c_hbm.at[wid], gsems[NBUF - 1]).wait()

    return pl.kernel(
        body_fn, mesh=mesh, out_type=out_type, scratch_types=scratch,
        compiler_params=pltpu.CompilerParams(
            needs_layout_passes=False, use_tc_tiling_on_sc=False))


RB = 1280


def _tc_k0(x_ref, w1_ref, xw_ref):
    xw_ref[...] = jnp.dot(x_ref[...], w1_ref[...],
                          preferred_element_type=jnp.float32)


def _tc_k1(degp_ref, xw_ref, dinv_ref, y1_ref):
    i = pl.program_id(0)
    iota = i * RB + lax.broadcasted_iota(jnp.int32, (1, RB), 1)
    valid = iota < N_NODES
    deg = degp_ref[0:1, :] + degp_ref[1:2, :] + 1.0
    dinv_row = jnp.where(valid, lax.rsqrt(deg), 0.0)
    dinv_ref[...] = dinv_row
    y1_ref[...] = dinv_row.reshape(RB, 1) * xw_ref[...]


def _tc_k2(p_ref, y_ref, dinv_ref, b_ref, w_ref, out_ref):
    dinv = dinv_ref[...].reshape(NPAD, 1)
    h = jax.nn.relu(dinv * (p_ref[0] + p_ref[1] + y_ref[...]) + b_ref[...])
    out_ref[...] = dinv * jnp.dot(h, w_ref[...],
                                  preferred_element_type=jnp.float32)


def _tc_k3(p_ref, y_ref, dinv_ref, b2_ref, cp_ref, w3_ref, b3_ref,
           lw_ref, lb_ref, out_ref):
    dinv = dinv_ref[...].reshape(NPAD, 1)
    h2 = jax.nn.relu(dinv * (p_ref[0] + p_ref[1] + y_ref[...]) + b2_ref[...])
    c = dinv * (jnp.sum(cp_ref[...], axis=1, keepdims=True) + dinv)
    ws = jnp.sum(c * h2, axis=0, keepdims=True)
    pooled = jnp.dot(ws, w3_ref[...],
                     preferred_element_type=jnp.float32) * (1.0 / N_NODES)
    pooled = pooled + b3_ref[...]
    logits = jnp.dot(pooled, lw_ref[...],
                     preferred_element_type=jnp.float32) + lb_ref[...]
    out_ref[...] = jax.nn.softmax(logits, axis=1)


def kernel(x, edge_index, W1, b1, W2, b2, W3, b3, lin_W, lin_b):
    n_edges = edge_index.shape[1]
    chunk = NW * EB * NBUF
    epad = -(-n_edges // chunk) * chunk
    kb = epad // (NW * EB)

    src = edge_index[0].astype(jnp.int32)
    dst = edge_index[1].astype(jnp.int32)
    pad_cfg = ((0, epad - n_edges),)
    srcr = jnp.pad(src, pad_cfg, constant_values=N_NODES).reshape(NW, kb, EB)
    dstr = jnp.pad(dst, pad_cfg, constant_values=N_NODES).reshape(NW, kb, EB)

    xw1 = pl.pallas_call(
        _tc_k0,
        grid=(NPAD // RB,),
        in_specs=[
            pl.BlockSpec((RB, x.shape[1]), lambda i: (i, 0)),
            pl.BlockSpec(W1.shape, lambda i: (0, 0)),
        ],
        out_specs=pl.BlockSpec((RB, H), lambda i: (i, 0)),
        out_shape=jax.ShapeDtypeStruct((NPAD, H), jnp.float32),
    )(x, W1)
    degp = _make_deg_kernel(kb)(dstr)

    dinv_c, y1 = pl.pallas_call(
        _tc_k1,
        grid=(NPAD // RB,),
        in_specs=[
            pl.BlockSpec((NC, RB), lambda i: (0, i)),
            pl.BlockSpec((RB, H), lambda i: (i, 0)),
        ],
        out_specs=(
            pl.BlockSpec((1, RB), lambda i: (0, i)),
            pl.BlockSpec((RB, H), lambda i: (i, 0)),
        ),
        out_shape=(
            jax.ShapeDtypeStruct((1, NPAD), jnp.float32),
            jax.ShapeDtypeStruct((NPAD, H), jnp.float32),
        ),
    )(degp, xw1)
    dinv_flat = dinv_c.reshape(NPAD)

    p1, cpart = _make_agg_kernel(kb, do_c=True)(y1, srcr, dstr, dinv_flat)

    y2 = pl.pallas_call(
        _tc_k2,
        out_shape=jax.ShapeDtypeStruct((NPAD, H), jnp.float32),
    )(p1, y1, dinv_c, b1.reshape(1, H), W2)

    (p2,) = _make_agg_kernel(kb, do_c=False)(y2, srcr, dstr)

    out = pl.pallas_call(
        _tc_k3,
        out_shape=jax.ShapeDtypeStruct((1, lin_W.shape[1]), jnp.float32),
    )(p2, y2, dinv_c, b2.reshape(1, H), cpart.T, W3,
      b3.reshape(1, H), lin_W, lin_b.reshape(1, lin_W.shape[1]))
    return out
